# Initial kernel scaffold; baseline (speedup 1.0000x reference)
import jax
import jax.numpy as jnp
from jax.experimental import pallas as pl


def kernel(x, edge_index, batch, W_f0, b_f0, W_f1, b_f1, W_c0, b_c0, W_c1, b_c1, W_node, b_node, W_edge, b_edge, W_b0, b_b0, W_b1, b_b1, W_pred, b_pred):
    raise NotImplementedError("write your pallas kernel here")


# speedup vs baseline: 9.3769x; 9.3769x over previous
import functools

import jax
import jax.numpy as jnp
from jax import lax
from jax.experimental import pallas as pl
from jax.experimental.pallas import tpu as pltpu
from jax.experimental.pallas import tpu_sc as plsc

N = 10000
NP = 10240
E = 320000
D = 128
G = 64
GP = 128

NC = 2
NS = 16
NW = NC * NS
C = 80

E_SC = E // NC
E_TILE = E_SC // NS
NCH = E_TILE // C
E_TILE_F = E // NS
NCH_F = E_TILE_F // C
RTS = NP // NS
RTW = NP // NW

_MESH = plsc.VectorSubcoreMesh(
    core_axis_name="c", subcore_axis_name="s", num_cores=NC, num_subcores=NS)
_SC_PARAMS = pltpu.CompilerParams(use_tc_tiling_on_sc=False)

_f32 = jnp.float32
_i32 = jnp.int32


def _fill(ref, rows, cols, val):
  v = jnp.full((16,), val, _f32)
  cg = cols // 16

  def body(i, carry):
    r = i // cg
    c = i % cg
    ref[r, pl.ds(c * 16, 16)] = v
    return carry

  lax.fori_loop(0, rows * cg, body, 0)


def _sc_stats(dst, batch):
  @functools.partial(
      pl.kernel, mesh=_MESH, compiler_params=_SC_PARAMS,
      out_type=(jax.ShapeDtypeStruct((NC, NP, 16), _f32),
                jax.ShapeDtypeStruct((NC, GP, 16), _f32)),
      scratch_types=[
          pltpu.VMEM((C,), _i32),
          pltpu.VMEM((C, 16), _f32),
          pltpu.VMEM_SHARED((NP, 16), _f32),
          pltpu.VMEM_SHARED((GP, 16), _f32),
      ])
  def k(dst_h, batch_h, deg_o, cnt_o, didx, buf, accd, accc):
    cid = lax.axis_index("c")
    sid = lax.axis_index("s")
    wid = cid * NS + sid
    _fill(buf, C, 16, 0.0)
    for j in range(RTS // C):
      pltpu.sync_copy(buf, accd.at[pl.ds(sid * RTS + j * C, C)])

    @pl.when(sid == 0)
    def _():
      pltpu.sync_copy(buf, accc.at[pl.ds(0, C)])
      pltpu.sync_copy(buf.at[pl.ds(0, GP - C)], accc.at[pl.ds(C, GP - C)])

    _fill(buf, C, 16, 1.0)
    plsc.subcore_barrier()

    ebase = cid * E_SC + sid * E_TILE

    def step(j, carry):
      pltpu.sync_copy(dst_h.at[pl.ds(ebase + j * C, C)], didx)
      pltpu.sync_copy(buf, accd.at[didx], add=True)
      return carry

    lax.fori_loop(0, NCH, step, 0)

    nbase = wid * RTW

    def nstep(j, carry):
      pltpu.sync_copy(batch_h.at[pl.ds(nbase + j * C, C)], didx)
      pltpu.sync_copy(buf, accc.at[didx], add=True)
      return carry

    lax.fori_loop(0, RTW // C, nstep, 0)

    plsc.subcore_barrier()
    pltpu.sync_copy(accd.at[pl.ds(sid * RTS, RTS)],
                    deg_o.at[cid, pl.ds(sid * RTS, RTS)])

    @pl.when(sid == 0)
    def _():
      pltpu.sync_copy(accc, cnt_o.at[cid])

  return k(dst, batch)


def _sc_spmm_half(g, src, dst, ec16=None):
  weighted = ec16 is not None
  scratch = [
      pltpu.VMEM((C,), _i32),
      pltpu.VMEM((C,), _i32),
      pltpu.VMEM((C, D), _f32),
      pltpu.VMEM((C, D), _f32),
      pltpu.SemaphoreType.DMA,
      pltpu.VMEM_SHARED((NP, D), _f32),
  ] + ([pltpu.VMEM((C, 16), _f32)] if weighted else [])

  @functools.partial(
      pl.kernel, mesh=_MESH, compiler_params=_SC_PARAMS,
      out_type=jax.ShapeDtypeStruct((NC, NP, D), _f32),
      scratch_types=scratch)
  def k(*args):
    if weighted:
      g_h, src_h, dst_h, ec_h, out_h, sidx, didx, rows, zbuf, sem, acc, ecv = args
    else:
      g_h, src_h, dst_h, out_h, sidx, didx, rows, zbuf, sem, acc = args
    cid = lax.axis_index("c")
    sid = lax.axis_index("s")
    _fill(zbuf, C, D, 0.0)
    for j in range(RTS // C):
      pltpu.sync_copy(zbuf, acc.at[pl.ds(sid * RTS + j * C, C)])
    plsc.subcore_barrier()

    ebase = cid * E_SC + sid * E_TILE

    def step(j, carry):
      off = ebase + j * C
      pltpu.sync_copy(src_h.at[pl.ds(off, C)], sidx)
      pltpu.sync_copy(dst_h.at[pl.ds(off, C)], didx)
      pltpu.async_copy(g_h.at[sidx], rows, sem).wait()
      if weighted:
        pltpu.sync_copy(ec_h.at[pl.ds(off, C)], ecv)

        def srow(r, c2):
          b = ecv[r, :]
          for q in range(D // 16):
            rows[r, pl.ds(q * 16, 16)] = rows[r, pl.ds(q * 16, 16)] * b
          return c2

        lax.fori_loop(0, C, srow, 0)
      pltpu.sync_copy(rows, acc.at[didx], add=True)
      return carry

    lax.fori_loop(0, NCH, step, 0)
    plsc.subcore_barrier()
    for j in range(RTS // C):
      r0 = sid * RTS + j * C
      pltpu.sync_copy(acc.at[pl.ds(r0, C)], out_h.at[cid, pl.ds(r0, C)])

  if weighted:
    return k(g, src, dst, ec16)
  return k(g, src, dst)


def _sc_spmm_dual(ga, gb, src, dst):
  @functools.partial(
      pl.kernel, mesh=_MESH, compiler_params=_SC_PARAMS,
      out_type=jax.ShapeDtypeStruct((NC, NP, D), _f32),
      scratch_types=[
          pltpu.VMEM((C,), _i32),
          pltpu.VMEM((C,), _i32),
          pltpu.VMEM((C, D), _f32),
          pltpu.VMEM((C, D), _f32),
          pltpu.SemaphoreType.DMA,
          pltpu.VMEM_SHARED((NP, D), _f32),
      ])
  def k(ga_h, gb_h, src_h, dst_h, out_h, sidx, didx, rows, zbuf, sem, acc):
    cid = lax.axis_index("c")
    sid = lax.axis_index("s")
    _fill(zbuf, C, D, 0.0)
    for j in range(RTS // C):
      pltpu.sync_copy(zbuf, acc.at[pl.ds(sid * RTS + j * C, C)])
    plsc.subcore_barrier()

    ebase = sid * E_TILE_F

    def loop(g_h):
      def step(j, carry):
        off = ebase + j * C
        pltpu.sync_copy(src_h.at[pl.ds(off, C)], sidx)
        pltpu.sync_copy(dst_h.at[pl.ds(off, C)], didx)
        pltpu.async_copy(g_h.at[sidx], rows, sem).wait()
        pltpu.sync_copy(rows, acc.at[didx], add=True)
        return carry

      lax.fori_loop(0, NCH_F, step, 0)

    @pl.when(cid == 0)
    def _():
      loop(ga_h)

    @pl.when(cid == 1)
    def _():
      loop(gb_h)

    plsc.subcore_barrier()
    for j in range(RTS // C):
      r0 = sid * RTS + j * C
      pltpu.sync_copy(acc.at[pl.ds(r0, C)], out_h.at[cid, pl.ds(r0, C)])

  return k(ga, gb, src, dst)


def _sc_edge(src, dst, p1b, p2b):
  @functools.partial(
      pl.kernel, mesh=_MESH, compiler_params=_SC_PARAMS,
      out_type=(jax.ShapeDtypeStruct((E, 16), _f32),
                jax.ShapeDtypeStruct((NC, NP, 16), _f32)),
      scratch_types=[
          pltpu.VMEM((C,), _i32),
          pltpu.VMEM((C,), _i32),
          pltpu.VMEM((C, 16), _f32),
          pltpu.VMEM((C, 16), _f32),
          pltpu.VMEM((C, 16), _f32),
          pltpu.SemaphoreType.DMA,
          pltpu.VMEM_SHARED((NP, 16), _f32),
      ])
  def k(src_h, dst_h, p1b_h, p2b_h, ec_o, degb_o,
        sidx, didx, ra, rb, ecv, sem, acc):
    cid = lax.axis_index("c")
    sid = lax.axis_index("s")
    _fill(ecv, C, 16, 0.0)
    for j in range(RTS // C):
      pltpu.sync_copy(ecv, acc.at[pl.ds(sid * RTS + j * C, C)])
    plsc.subcore_barrier()

    ebase = cid * E_SC + sid * E_TILE

    def step(j, carry):
      off = ebase + j * C
      pltpu.sync_copy(src_h.at[pl.ds(off, C)], sidx)
      pltpu.sync_copy(dst_h.at[pl.ds(off, C)], didx)
      pltpu.async_copy(p1b_h.at[sidx], ra, sem).wait()
      pltpu.async_copy(p2b_h.at[didx], rb, sem).wait()

      def erow(r, c2):
        z = ra[r, :] + rb[r, :]
        ecv[r, :] = 1.0 / (1.0 + jnp.exp(-z))
        return c2

      lax.fori_loop(0, C, erow, 0)
      pltpu.sync_copy(ecv, acc.at[didx], add=True)
      pltpu.sync_copy(ecv, ec_o.at[pl.ds(off, C)])
      return carry

    lax.fori_loop(0, NCH, step, 0)
    plsc.subcore_barrier()
    pltpu.sync_copy(acc.at[pl.ds(sid * RTS, RTS)],
                    degb_o.at[cid, pl.ds(sid * RTS, RTS)])

  return k(src, dst, p1b, p2b)


def _sc_pool(h, batch):
  @functools.partial(
      pl.kernel, mesh=_MESH, compiler_params=_SC_PARAMS,
      out_type=jax.ShapeDtypeStruct((NC, GP, D), _f32),
      scratch_types=[
          pltpu.VMEM((C,), _i32),
          pltpu.VMEM((C, D), _f32),
          pltpu.VMEM((C, D), _f32),
          pltpu.VMEM_SHARED((GP, D), _f32),
      ])
  def k(h_h, batch_h, out_h, bidx, rows, zbuf, acc):
    cid = lax.axis_index("c")
    sid = lax.axis_index("s")
    wid = cid * NS + sid

    @pl.when(sid == 0)
    def _():
      _fill(zbuf, C, D, 0.0)
      pltpu.sync_copy(zbuf, acc.at[pl.ds(0, C)])
      pltpu.sync_copy(zbuf.at[pl.ds(0, GP - C)], acc.at[pl.ds(C, GP - C)])

    plsc.subcore_barrier()
    nbase = wid * RTW

    def step(j, carry):
      off = nbase + j * C
      pltpu.sync_copy(batch_h.at[pl.ds(off, C)], bidx)
      pltpu.sync_copy(h_h.at[pl.ds(off, C)], rows)
      pltpu.sync_copy(rows, acc.at[bidx], add=True)
      return carry

    lax.fori_loop(0, RTW // C, step, 0)
    plsc.subcore_barrier()

    @pl.when(sid == 0)
    def _():
      pltpu.sync_copy(acc, out_h.at[cid])

  return k(h, batch)


_RB = 512
_GRID = NP // _RB


def _rspec(shape3=False, cols=D):
  if shape3:
    return pl.BlockSpec((NC, _RB, cols), lambda j: (0, j, 0))
  return pl.BlockSpec((_RB, cols), lambda j: (j, 0))


def _wspec(r, c):
  return pl.BlockSpec((r, c), lambda j: (0, 0))


def _tc_prep(deg2, x):
  def body(d_r, x_r, q_r, g_r):
    deg = d_r[0, :, 0:1] + d_r[1, :, 0:1] + 1.0
    q = lax.rsqrt(deg)
    q_r[...] = q
    g_r[...] = q * x_r[...]

  return pl.pallas_call(
      body, grid=(_GRID,),
      in_specs=[_rspec(True, 16), _rspec()],
      out_specs=[_rspec(cols=1), _rspec()],
      out_shape=[jax.ShapeDtypeStruct((NP, 1), _f32),
                 jax.ShapeDtypeStruct((NP, D), _f32)])(deg2, x)


def _tc_conv2(S0, g0, q, Wf, bf, Wc, bc):
  def body(s_r, g_r, q_r, wf_r, bf_r, wc_r, bc_r, of_r, oc_r):
    t = q_r[...] * (s_r[0] + s_r[1] + g_r[...])
    hf = jax.nn.relu(jnp.dot(t, wf_r[...], preferred_element_type=_f32)
                     + bf_r[...])
    hc = jax.nn.relu(jnp.dot(t, wc_r[...], preferred_element_type=_f32)
                     + bc_r[...])
    of_r[...] = q_r[...] * hf
    oc_r[...] = q_r[...] * hc

  return pl.pallas_call(
      body, grid=(_GRID,),
      in_specs=[_rspec(True), _rspec(), _rspec(cols=1),
                _wspec(D, D), _wspec(1, D), _wspec(D, D), _wspec(1, D)],
      out_specs=[_rspec(), _rspec()],
      out_shape=[jax.ShapeDtypeStruct((NP, D), _f32),
                 jax.ShapeDtypeStruct((NP, D), _f32)])(
                     S0, g0, q, Wf, bf, Wc, bc)


def _tc_caus(S1, g1f, g1c, q, Wf1, bf1, Wc1, bc1, Wsm, bsm):
  def body(s_r, gf_r, gc_r, q_r, wf_r, bf_r, wc_r, bc_r, wsm_r, bsm_r,
           xe_r, nc_r, p1_r, p2_r):
    q = q_r[...]
    xe_r[...] = jax.nn.relu(
        jnp.dot(q * (s_r[0] + gf_r[...]), wf_r[...],
                preferred_element_type=_f32) + bf_r[...])
    h2c = jax.nn.relu(
        jnp.dot(q * (s_r[1] + gc_r[...]), wc_r[...],
                preferred_element_type=_f32) + bc_r[...])
    sm = jnp.dot(h2c, wsm_r[...], preferred_element_type=_f32) + bsm_r[...]
    nc_r[...] = jax.nn.sigmoid(sm[:, 0:1])
    p1_r[...] = jnp.broadcast_to(sm[:, 1:2], (_RB, 16))
    p2_r[...] = jnp.broadcast_to(sm[:, 2:3], (_RB, 16))

  return pl.pallas_call(
      body, grid=(_GRID,),
      in_specs=[_rspec(True), _rspec(), _rspec(), _rspec(cols=1),
                _wspec(D, D), _wspec(1, D), _wspec(D, D), _wspec(1, D),
                _wspec(D, 3), _wspec(1, 3)],
      out_specs=[_rspec(), _rspec(cols=1), _rspec(cols=16), _rspec(cols=16)],
      out_shape=[jax.ShapeDtypeStruct((NP, D), _f32),
                 jax.ShapeDtypeStruct((NP, 1), _f32),
                 jax.ShapeDtypeStruct((NP, 16), _f32),
                 jax.ShapeDtypeStruct((NP, 16), _f32)])(
                     S1, g1f, g1c, q, Wf1, bf1, Wc1, bc1, Wsm, bsm)


def _tc_prep2(degb2, xe, nc):
  def body(d_r, xe_r, nc_r, qb_r, gb_r):
    qb = lax.rsqrt(d_r[0, :, 0:1] + d_r[1, :, 0:1] + 1.0)
    qb_r[...] = qb
    gb_r[...] = qb * (xe_r[...] * nc_r[...])

  return pl.pallas_call(
      body, grid=(_GRID,),
      in_specs=[_rspec(True, 16), _rspec(), _rspec(cols=1)],
      out_specs=[_rspec(cols=1), _rspec()],
      out_shape=[jax.ShapeDtypeStruct((NP, 1), _f32),
                 jax.ShapeDtypeStruct((NP, D), _f32)])(degb2, xe, nc)


def _tc_conv1(Sb, gb, qb, nc, W, b):
  def body(s_r, g_r, q_r, nc_r, w_r, b_r, o_r):
    h = jax.nn.relu(
        jnp.dot(q_r[...] * (s_r[0] + s_r[1] + g_r[...]), w_r[...],
                preferred_element_type=_f32) + b_r[...])
    o_r[...] = q_r[...] * (h * nc_r[...])

  return pl.pallas_call(
      body, grid=(_GRID,),
      in_specs=[_rspec(True), _rspec(), _rspec(cols=1), _rspec(cols=1),
                _wspec(D, D), _wspec(1, D)],
      out_specs=_rspec(),
      out_shape=jax.ShapeDtypeStruct((NP, D), _f32))(Sb, gb, qb, nc, W, b)


def _tc_conv_last(Sb, gb, qb, W, b):
  def body(s_r, g_r, q_r, w_r, b_r, o_r):
    o_r[...] = jax.nn.relu(
        jnp.dot(q_r[...] * (s_r[0] + s_r[1] + g_r[...]), w_r[...],
                preferred_element_type=_f32) + b_r[...])

  return pl.pallas_call(
      body, grid=(_GRID,),
      in_specs=[_rspec(True), _rspec(), _rspec(cols=1),
                _wspec(D, D), _wspec(1, D)],
      out_specs=_rspec(),
      out_shape=jax.ShapeDtypeStruct((NP, D), _f32))(Sb, gb, qb, W, b)


def _tc_pred(pooled, cnt2, W_pred, b_pred):
  def body(p_r, c_r, w_r, b_r, o_r):
    sums = p_r[0, :G, :] + p_r[1, :G, :]
    cnt = c_r[0, :G, 0:1] + c_r[1, :G, 0:1]
    hg = sums / jnp.maximum(cnt, 1.0)
    o_r[...] = jnp.dot(hg, w_r[...], preferred_element_type=_f32) + b_r[...]

  return pl.pallas_call(
      body,
      out_shape=jax.ShapeDtypeStruct((G, 10), _f32))(
          pooled, cnt2, W_pred, b_pred)


def kernel(x, edge_index, batch, W_f0, b_f0, W_f1, b_f1, W_c0, b_c0, W_c1,
           b_c1, W_node, b_node, W_edge, b_edge, W_b0, b_b0, W_b1, b_b1,
           W_pred, b_pred):
  src = edge_index[0].astype(_i32)
  dst = edge_index[1].astype(_i32)
  xp = jnp.pad(x, ((0, NP - N), (0, 0)))
  batchp = jnp.pad(batch.astype(_i32), (0, NP - N), constant_values=G)

  deg2, cnt2 = _sc_stats(dst, batchp)
  q, g0 = _tc_prep(deg2, xp)
  S0 = _sc_spmm_half(g0, src, dst)
  g1f, g1c = _tc_conv2(S0, g0, q,
                       W_f0, b_f0.reshape(1, D), W_c0, b_c0.reshape(1, D))
  S1 = _sc_spmm_dual(g1f, g1c, src, dst)
  Wsm = jnp.concatenate([W_node, W_edge[:D], W_edge[D:]], axis=1)
  bsm = jnp.stack([b_node[0], jnp.zeros((), _f32), b_edge[0]]).reshape(1, 3)
  xe, nc, p1b, p2b = _tc_caus(S1, g1f, g1c, q,
                              W_f1, b_f1.reshape(1, D), W_c1,
                              b_c1.reshape(1, D), Wsm, bsm)
  ec16, degb2 = _sc_edge(src, dst, p1b, p2b)
  qb, gb0 = _tc_prep2(degb2, xe, nc)
  Sb0 = _sc_spmm_half(gb0, src, dst, ec16)
  gb1 = _tc_conv1(Sb0, gb0, qb, nc, W_b0, b_b0.reshape(1, D))
  Sb1 = _sc_spmm_half(gb1, src, dst, ec16)
  h2 = _tc_conv_last(Sb1, gb1, qb, W_b1, b_b1.reshape(1, D))
  pooled = _sc_pool(h2, batchp)
  return _tc_pred(pooled, cnt2, W_pred, b_pred.reshape(1, 10))


# speedup vs baseline: 16.3729x; 1.7461x over previous
import functools

import jax
import jax.numpy as jnp
from jax import lax
from jax.experimental import pallas as pl
from jax.experimental.pallas import tpu as pltpu
from jax.experimental.pallas import tpu_sc as plsc

N = 10000
NP = 10240
E = 320000
D = 128
G = 64
GP = 128

NC = 2
NS = 16
NW = NC * NS
C = 40

E_SC = E // NC
E_TILE = E_SC // NS
NCH = E_TILE // C
E_TILE_F = E // NS
NCH_F = E_TILE_F // C
RTS = NP // NS
RTW = NP // NW

_MESH = plsc.VectorSubcoreMesh(
    core_axis_name="c", subcore_axis_name="s", num_cores=NC, num_subcores=NS)
_SC_PARAMS = pltpu.CompilerParams(use_tc_tiling_on_sc=False)

_f32 = jnp.float32
_i32 = jnp.int32


def _zero_rows(acc, zbuf, total):
  off = 0
  while off < total:
    n = min(C, total - off)
    pltpu.sync_copy(zbuf.at[pl.ds(0, n)], acc.at[pl.ds(off, n)])
    off += n


def _fill(ref, rows, cols, val):
  v = jnp.full((16,), val, _f32)
  cg = cols // 16

  def body(i, carry):
    r = i // cg
    c = i % cg
    ref[r, pl.ds(c * 16, 16)] = v
    return carry

  lax.fori_loop(0, rows * cg, body, 0)


def _sc_stats(dst, batch):
  @functools.partial(
      pl.kernel, mesh=_MESH, compiler_params=_SC_PARAMS,
      out_type=(jax.ShapeDtypeStruct((NC, NP, 16), _f32),
                jax.ShapeDtypeStruct((NC, GP, 16), _f32)),
      scratch_types=[
          pltpu.VMEM((C,), _i32),
          pltpu.VMEM((C, 16), _f32),
          pltpu.VMEM_SHARED((NP, 16), _f32),
          pltpu.VMEM_SHARED((GP, 16), _f32),
      ])
  def k(dst_h, batch_h, deg_o, cnt_o, didx, buf, accd, accc):
    cid = lax.axis_index("c")
    sid = lax.axis_index("s")
    wid = cid * NS + sid
    _fill(buf, C, 16, 0.0)
    for j in range(RTS // C):
      pltpu.sync_copy(buf, accd.at[pl.ds(sid * RTS + j * C, C)])

    @pl.when(sid == 0)
    def _():
      _zero_rows(accc, buf, GP)

    _fill(buf, C, 16, 1.0)
    plsc.subcore_barrier()

    ebase = cid * E_SC + sid * E_TILE

    def step(j, carry):
      pltpu.sync_copy(dst_h.at[pl.ds(ebase + j * C, C)], didx)
      pltpu.sync_copy(buf, accd.at[didx], add=True)
      return carry

    lax.fori_loop(0, NCH, step, 0)

    nbase = wid * RTW

    def nstep(j, carry):
      pltpu.sync_copy(batch_h.at[pl.ds(nbase + j * C, C)], didx)
      pltpu.sync_copy(buf, accc.at[didx], add=True)
      return carry

    lax.fori_loop(0, RTW // C, nstep, 0)

    plsc.subcore_barrier()
    pltpu.sync_copy(accd.at[pl.ds(sid * RTS, RTS)],
                    deg_o.at[cid, pl.ds(sid * RTS, RTS)])

    @pl.when(sid == 0)
    def _():
      pltpu.sync_copy(accc, cnt_o.at[cid])

  return k(dst, batch)


_NB = 4
_LA = 2


def _pipe(g_h, sidx_all, didx_all, rows, gsems, ssems, acc, nch,
          scale_fn=None):
  for b in range(_LA):
    pltpu.async_copy(g_h.at[sidx_all.at[b]], rows.at[b], gsems[b])

  def step(p, carry):
    for b in range(_NB):
      j = p * _NB + b

      @pl.when(j < nch)
      def _(j=j, b=b):
        pltpu.make_async_copy(
            g_h.at[sidx_all.at[j]], rows.at[b], gsems[b]).wait()
        if scale_fn is not None:
          scale_fn(j, b)
        pltpu.async_copy(rows.at[b], acc.at[didx_all.at[j]], ssems[b],
                         add=True)

      jn = j + _LA
      bn = (b + _LA) % _NB

      @pl.when(jnp.logical_and(jn < nch, jn >= _NB))
      def _(jn=jn, bn=bn):
        pltpu.make_async_copy(
            rows.at[bn], acc.at[didx_all.at[jn - _NB]], ssems[bn]).wait()
        pltpu.async_copy(g_h.at[sidx_all.at[jn]], rows.at[bn], gsems[bn])

      @pl.when(jnp.logical_and(jn < nch, jn < _NB))
      def _(jn=jn, bn=bn):
        pltpu.async_copy(g_h.at[sidx_all.at[jn]], rows.at[bn], gsems[bn])

    return carry

  lax.fori_loop(0, (nch + _NB - 1) // _NB, step, 0)
  for kk in range(_NB):
    c = nch - _NB + kk
    pltpu.make_async_copy(
        rows.at[c % _NB], acc.at[didx_all.at[c]], ssems[c % _NB]).wait()


def _spmm_scratch(nch):
  return [
      pltpu.VMEM((nch, C), _i32),
      pltpu.VMEM((nch, C), _i32),
      pltpu.VMEM((_NB, C, D), _f32),
      pltpu.VMEM((C, D), _f32),
  ] + [pltpu.SemaphoreType.DMA] * (2 * _NB) + [
      pltpu.VMEM_SHARED((NP, D), _f32),
  ]


def _sc_spmm_half(g, src2, dst2, ec16=None):
  weighted = ec16 is not None
  scratch = _spmm_scratch(NCH) + (
      [pltpu.VMEM((C, 16), _f32)] if weighted else [])

  @functools.partial(
      pl.kernel, mesh=_MESH, compiler_params=_SC_PARAMS,
      out_type=jax.ShapeDtypeStruct((NC, NP, D), _f32),
      scratch_types=scratch)
  def k(*args):
    if weighted:
      (g_h, src_h, dst_h, ec_h, out_h, sidx_all, didx_all, rows, zbuf,
       *sems, acc, ecv) = args
    else:
      (g_h, src_h, dst_h, out_h, sidx_all, didx_all, rows, zbuf,
       *sems, acc) = args
    gsems, ssems = sems[:_NB], sems[_NB:2 * _NB]
    cid = lax.axis_index("c")
    sid = lax.axis_index("s")
    _fill(zbuf, C, D, 0.0)
    for j in range(RTS // C):
      pltpu.sync_copy(zbuf, acc.at[pl.ds(sid * RTS + j * C, C)])
    cbase = (cid * E_SC + sid * E_TILE) // C
    pltpu.sync_copy(src_h.at[pl.ds(cbase, NCH)], sidx_all)
    pltpu.sync_copy(dst_h.at[pl.ds(cbase, NCH)], didx_all)
    plsc.subcore_barrier()

    if weighted:
      ebase = cid * E_SC + sid * E_TILE

      def scale(j, b):
        pltpu.sync_copy(ec_h.at[pl.ds(ebase + j * C, C)], ecv)

        def srow(r, c2):
          ev = ecv[r, :]
          for q in range(D // 16):
            rows[b, r, pl.ds(q * 16, 16)] = rows[b, r, pl.ds(q * 16, 16)] * ev
          return c2

        lax.fori_loop(0, C, srow, 0)
    else:
      scale = None

    _pipe(g_h, sidx_all, didx_all, rows, gsems, ssems, acc, NCH, scale)
    plsc.subcore_barrier()
    for j in range(RTS // C):
      r0 = sid * RTS + j * C
      pltpu.sync_copy(acc.at[pl.ds(r0, C)], out_h.at[cid, pl.ds(r0, C)])

  if weighted:
    return k(g, src2, dst2, ec16)
  return k(g, src2, dst2)


def _sc_edge(src2, dst2, p1b, p2b):
  @functools.partial(
      pl.kernel, mesh=_MESH, compiler_params=_SC_PARAMS,
      out_type=(jax.ShapeDtypeStruct((E, 16), _f32),
                jax.ShapeDtypeStruct((NC, NP, 16), _f32)),
      scratch_types=[
          pltpu.VMEM((NCH, C), _i32),
          pltpu.VMEM((NCH, C), _i32),
          pltpu.VMEM((_NB, C, 16), _f32),
          pltpu.VMEM((_NB, C, 16), _f32),
          pltpu.VMEM((_NB, C, 16), _f32),
          pltpu.VMEM((C, 16), _f32),
      ] + [pltpu.SemaphoreType.DMA] * (2 * _NB) + [
          pltpu.VMEM_SHARED((NP, 16), _f32),
      ])
  def k(src_h, dst_h, p1b_h, p2b_h, ec_o, degb_o,
        sidx_all, didx_all, ra, rb, ecv, zbuf, *sems_acc):
    gsems = sems_acc[:_NB]
    ssems = sems_acc[_NB:2 * _NB]
    acc = sems_acc[2 * _NB]
    cid = lax.axis_index("c")
    sid = lax.axis_index("s")
    _fill(zbuf, C, 16, 0.0)
    for j in range(RTS // C):
      pltpu.sync_copy(zbuf, acc.at[pl.ds(sid * RTS + j * C, C)])
    cbase = (cid * E_SC + sid * E_TILE) // C
    pltpu.sync_copy(src_h.at[pl.ds(cbase, NCH)], sidx_all)
    pltpu.sync_copy(dst_h.at[pl.ds(cbase, NCH)], didx_all)
    plsc.subcore_barrier()

    ebase = cid * E_SC + sid * E_TILE

    def start_gather(j, b):
      pltpu.async_copy(p1b_h.at[sidx_all.at[j]], ra.at[b], gsems[b])
      pltpu.async_copy(p2b_h.at[didx_all.at[j]], rb.at[b], gsems[b])

    for b in range(_LA):
      start_gather(b, b)

    def step(p, carry):
      for b in range(_NB):
        j = p * _NB + b

        @pl.when(j < NCH)
        def _(j=j, b=b):
          pltpu.make_async_copy(
              p1b_h.at[sidx_all.at[j]], ra.at[b], gsems[b]).wait()
          pltpu.make_async_copy(
              p2b_h.at[didx_all.at[j]], rb.at[b], gsems[b]).wait()

          @pl.when(j >= _NB)
          def _():
            pltpu.make_async_copy(
                ecv.at[b], acc.at[didx_all.at[j - _NB]], ssems[b]).wait()

          def erow(r, c2):
            z = ra[b, r, :] + rb[b, r, :]
            ecv[b, r, :] = 1.0 / (1.0 + jnp.exp(-z))
            return c2

          lax.fori_loop(0, C, erow, 0)
          pltpu.async_copy(ecv.at[b], acc.at[didx_all.at[j]], ssems[b],
                           add=True)
          pltpu.sync_copy(ecv.at[b], ec_o.at[pl.ds(ebase + j * C, C)])

        jn = j + _LA
        bn = (b + _LA) % _NB

        @pl.when(jn < NCH)
        def _(jn=jn, bn=bn):
          start_gather(jn, bn)

      return carry

    lax.fori_loop(0, (NCH + _NB - 1) // _NB, step, 0)
    for kk in range(_NB):
      c = NCH - _NB + kk
      pltpu.make_async_copy(
          ecv.at[c % _NB], acc.at[didx_all.at[c]], ssems[c % _NB]).wait()
    plsc.subcore_barrier()
    pltpu.sync_copy(acc.at[pl.ds(sid * RTS, RTS)],
                    degb_o.at[cid, pl.ds(sid * RTS, RTS)])

  return k(src2, dst2, p1b, p2b)


def _sc_pool(h, batch):
  @functools.partial(
      pl.kernel, mesh=_MESH, compiler_params=_SC_PARAMS,
      out_type=jax.ShapeDtypeStruct((NC, GP, D), _f32),
      scratch_types=[
          pltpu.VMEM((C,), _i32),
          pltpu.VMEM((C, D), _f32),
          pltpu.VMEM((C, D), _f32),
          pltpu.VMEM_SHARED((GP, D), _f32),
      ])
  def k(h_h, batch_h, out_h, bidx, rows, zbuf, acc):
    cid = lax.axis_index("c")
    sid = lax.axis_index("s")
    wid = cid * NS + sid

    @pl.when(sid == 0)
    def _():
      _fill(zbuf, C, D, 0.0)
      _zero_rows(acc, zbuf, GP)

    plsc.subcore_barrier()
    nbase = wid * RTW

    def step(j, carry):
      off = nbase + j * C
      pltpu.sync_copy(batch_h.at[pl.ds(off, C)], bidx)
      pltpu.sync_copy(h_h.at[pl.ds(off, C)], rows)
      pltpu.sync_copy(rows, acc.at[bidx], add=True)
      return carry

    lax.fori_loop(0, RTW // C, step, 0)
    plsc.subcore_barrier()

    @pl.when(sid == 0)
    def _():
      pltpu.sync_copy(acc, out_h.at[cid])

  return k(h, batch)


_RB = 512
_GRID = NP // _RB


def _rspec(shape3=False, cols=D):
  if shape3:
    return pl.BlockSpec((NC, _RB, cols), lambda j: (0, j, 0))
  return pl.BlockSpec((_RB, cols), lambda j: (j, 0))


def _wspec(r, c):
  return pl.BlockSpec((r, c), lambda j: (0, 0))


def _tc_prep(deg2, x):
  def body(d_r, x_r, q_r, g_r):
    deg = d_r[0, :, 0:1] + d_r[1, :, 0:1] + 1.0
    q = lax.rsqrt(deg)
    q_r[...] = q
    g_r[...] = q * x_r[...]

  return pl.pallas_call(
      body, grid=(_GRID,),
      in_specs=[_rspec(True, 16), _rspec()],
      out_specs=[_rspec(cols=1), _rspec()],
      out_shape=[jax.ShapeDtypeStruct((NP, 1), _f32),
                 jax.ShapeDtypeStruct((NP, D), _f32)])(deg2, x)


def _tc_conv2(S0, g0, q, Wf, bf, Wc, bc):
  def body(s_r, g_r, q_r, wf_r, bf_r, wc_r, bc_r, of_r, oc_r):
    t = q_r[...] * (s_r[0] + s_r[1] + g_r[...])
    hf = jax.nn.relu(jnp.dot(t, wf_r[...], preferred_element_type=_f32)
                     + bf_r[...])
    hc = jax.nn.relu(jnp.dot(t, wc_r[...], preferred_element_type=_f32)
                     + bc_r[...])
    of_r[...] = q_r[...] * hf
    oc_r[...] = q_r[...] * hc

  return pl.pallas_call(
      body, grid=(_GRID,),
      in_specs=[_rspec(True), _rspec(), _rspec(cols=1),
                _wspec(D, D), _wspec(1, D), _wspec(D, D), _wspec(1, D)],
      out_specs=[_rspec(), _rspec()],
      out_shape=[jax.ShapeDtypeStruct((NP, D), _f32),
                 jax.ShapeDtypeStruct((NP, D), _f32)])(
                     S0, g0, q, Wf, bf, Wc, bc)


def _tc_caus(S1f, S1c, g1f, g1c, q, Wf1, bf1, Wc1, bc1, Wsm, bsm):
  def body(sf_r, sc_r, gf_r, gc_r, q_r, wf_r, bf_r, wc_r, bc_r, wsm_r, bsm_r,
           xe_r, nc_r, p1_r, p2_r):
    q = q_r[...]
    xe_r[...] = jax.nn.relu(
        jnp.dot(q * (sf_r[0] + sf_r[1] + gf_r[...]), wf_r[...],
                preferred_element_type=_f32) + bf_r[...])
    h2c = jax.nn.relu(
        jnp.dot(q * (sc_r[0] + sc_r[1] + gc_r[...]), wc_r[...],
                preferred_element_type=_f32) + bc_r[...])
    sm = jnp.dot(h2c, wsm_r[...], preferred_element_type=_f32) + bsm_r[...]
    nc_r[...] = jax.nn.sigmoid(sm[:, 0:1])
    p1_r[...] = jnp.broadcast_to(sm[:, 1:2], (_RB, 16))
    p2_r[...] = jnp.broadcast_to(sm[:, 2:3], (_RB, 16))

  return pl.pallas_call(
      body, grid=(_GRID,),
      in_specs=[_rspec(True), _rspec(True), _rspec(), _rspec(),
                _rspec(cols=1),
                _wspec(D, D), _wspec(1, D), _wspec(D, D), _wspec(1, D),
                _wspec(D, 3), _wspec(1, 3)],
      out_specs=[_rspec(), _rspec(cols=1), _rspec(cols=16), _rspec(cols=16)],
      out_shape=[jax.ShapeDtypeStruct((NP, D), _f32),
                 jax.ShapeDtypeStruct((NP, 1), _f32),
                 jax.ShapeDtypeStruct((NP, 16), _f32),
                 jax.ShapeDtypeStruct((NP, 16), _f32)])(
                     S1f, S1c, g1f, g1c, q, Wf1, bf1, Wc1, bc1, Wsm, bsm)


def _tc_prep2(degb2, xe, nc):
  def body(d_r, xe_r, nc_r, qb_r, gb_r):
    qb = lax.rsqrt(d_r[0, :, 0:1] + d_r[1, :, 0:1] + 1.0)
    qb_r[...] = qb
    gb_r[...] = qb * (xe_r[...] * nc_r[...])

  return pl.pallas_call(
      body, grid=(_GRID,),
      in_specs=[_rspec(True, 16), _rspec(), _rspec(cols=1)],
      out_specs=[_rspec(cols=1), _rspec()],
      out_shape=[jax.ShapeDtypeStruct((NP, 1), _f32),
                 jax.ShapeDtypeStruct((NP, D), _f32)])(degb2, xe, nc)


def _tc_conv1(Sb, gb, qb, nc, W, b):
  def body(s_r, g_r, q_r, nc_r, w_r, b_r, o_r):
    h = jax.nn.relu(
        jnp.dot(q_r[...] * (s_r[0] + s_r[1] + g_r[...]), w_r[...],
                preferred_element_type=_f32) + b_r[...])
    o_r[...] = q_r[...] * (h * nc_r[...])

  return pl.pallas_call(
      body, grid=(_GRID,),
      in_specs=[_rspec(True), _rspec(), _rspec(cols=1), _rspec(cols=1),
                _wspec(D, D), _wspec(1, D)],
      out_specs=_rspec(),
      out_shape=jax.ShapeDtypeStruct((NP, D), _f32))(Sb, gb, qb, nc, W, b)


def _tc_conv_last(Sb, gb, qb, W, b):
  def body(s_r, g_r, q_r, w_r, b_r, o_r):
    o_r[...] = jax.nn.relu(
        jnp.dot(q_r[...] * (s_r[0] + s_r[1] + g_r[...]), w_r[...],
                preferred_element_type=_f32) + b_r[...])

  return pl.pallas_call(
      body, grid=(_GRID,),
      in_specs=[_rspec(True), _rspec(), _rspec(cols=1),
                _wspec(D, D), _wspec(1, D)],
      out_specs=_rspec(),
      out_shape=jax.ShapeDtypeStruct((NP, D), _f32))(Sb, gb, qb, W, b)


def _tc_pred(pooled, cnt2, W_pred, b_pred):
  def body(p_r, c_r, w_r, b_r, o_r):
    sums = p_r[0, :G, :] + p_r[1, :G, :]
    cnt = c_r[0, :G, 0:1] + c_r[1, :G, 0:1]
    hg = sums / jnp.maximum(cnt, 1.0)
    o_r[...] = jnp.dot(hg, w_r[...], preferred_element_type=_f32) + b_r[...]

  return pl.pallas_call(
      body,
      out_shape=jax.ShapeDtypeStruct((G, 10), _f32))(
          pooled, cnt2, W_pred, b_pred)


def kernel(x, edge_index, batch, W_f0, b_f0, W_f1, b_f1, W_c0, b_c0, W_c1,
           b_c1, W_node, b_node, W_edge, b_edge, W_b0, b_b0, W_b1, b_b1,
           W_pred, b_pred):
  src = edge_index[0].astype(_i32)
  dst = edge_index[1].astype(_i32)
  src2 = src.reshape(E // C, C)
  dst2 = dst.reshape(E // C, C)
  xp = jnp.pad(x, ((0, NP - N), (0, 0)))
  batchp = jnp.pad(batch.astype(_i32), (0, NP - N), constant_values=G)

  deg2, cnt2 = _sc_stats(dst, batchp)
  q, g0 = _tc_prep(deg2, xp)
  S0 = _sc_spmm_half(g0, src2, dst2)
  g1f, g1c = _tc_conv2(S0, g0, q,
                       W_f0, b_f0.reshape(1, D), W_c0, b_c0.reshape(1, D))
  S1f = _sc_spmm_half(g1f, src2, dst2)
  S1c = _sc_spmm_half(g1c, src2, dst2)
  Wsm = jnp.concatenate([W_node, W_edge[:D], W_edge[D:]], axis=1)
  bsm = jnp.stack([b_node[0], jnp.zeros((), _f32), b_edge[0]]).reshape(1, 3)
  xe, nc, p1b, p2b = _tc_caus(S1f, S1c, g1f, g1c, q,
                              W_f1, b_f1.reshape(1, D), W_c1,
                              b_c1.reshape(1, D), Wsm, bsm)
  ec16, degb2 = _sc_edge(src2, dst2, p1b, p2b)
  qb, gb0 = _tc_prep2(degb2, xe, nc)
  Sb0 = _sc_spmm_half(gb0, src2, dst2, ec16)
  gb1 = _tc_conv1(Sb0, gb0, qb, nc, W_b0, b_b0.reshape(1, D))
  Sb1 = _sc_spmm_half(gb1, src2, dst2, ec16)
  h2 = _tc_conv_last(Sb1, gb1, qb, W_b1, b_b1.reshape(1, D))
  pooled = _sc_pool(h2, batchp)
  return _tc_pred(pooled, cnt2, W_pred, b_pred.reshape(1, 10))


# speedup vs baseline: 18.3726x; 1.1221x over previous
import functools

import jax
import jax.numpy as jnp
from jax import lax
from jax.experimental import pallas as pl
from jax.experimental.pallas import tpu as pltpu
from jax.experimental.pallas import tpu_sc as plsc

N = 10000
NP = 10240
E = 320000
D = 128
G = 64
GP = 128

NC = 2
NS = 16
NW = NC * NS
C = 40

E_SC = E // NC
E_TILE = E_SC // NS
NCH = E_TILE // C
E_TILE_F = E // NS
NCH_F = E_TILE_F // C
RTS = NP // NS
RTW = NP // NW

_MESH = plsc.VectorSubcoreMesh(
    core_axis_name="c", subcore_axis_name="s", num_cores=NC, num_subcores=NS)
_SC_PARAMS = pltpu.CompilerParams(use_tc_tiling_on_sc=False)

_f32 = jnp.float32
_i32 = jnp.int32


def _zero_rows(acc, zbuf, total):
  off = 0
  while off < total:
    n = min(C, total - off)
    pltpu.sync_copy(zbuf.at[pl.ds(0, n)], acc.at[pl.ds(off, n)])
    off += n


def _fill(ref, rows, cols, val):
  v = jnp.full((16,), val, _f32)
  cg = cols // 16

  def body(i, carry):
    r = i // cg
    c = i % cg
    ref[r, pl.ds(c * 16, 16)] = v
    return carry

  lax.fori_loop(0, rows * cg, body, 0)


def _sc_stats(dst, batch):
  @functools.partial(
      pl.kernel, mesh=_MESH, compiler_params=_SC_PARAMS,
      out_type=(jax.ShapeDtypeStruct((NC, NP, 16), _f32),
                jax.ShapeDtypeStruct((NC, GP, 16), _f32)),
      scratch_types=[
          pltpu.VMEM((NCH, C), _i32),
          pltpu.VMEM((RTW // C, C), _i32),
          pltpu.VMEM((C, 16), _f32),
      ] + [pltpu.SemaphoreType.DMA] * _NB + [
          pltpu.VMEM_SHARED((NP, 16), _f32),
          pltpu.VMEM_SHARED((GP, 16), _f32),
      ])
  def k(dst_h, batch_h, deg_o, cnt_o, didx_all, bidx_all, buf, *sems_acc):
    ssems = sems_acc[:_NB]
    accd, accc = sems_acc[_NB], sems_acc[_NB + 1]
    cid = lax.axis_index("c")
    sid = lax.axis_index("s")
    wid = cid * NS + sid
    _fill(buf, C, 16, 0.0)
    for j in range(RTS // C):
      pltpu.sync_copy(buf, accd.at[pl.ds(sid * RTS + j * C, C)])

    @pl.when(sid == 0)
    def _():
      _zero_rows(accc, buf, GP)

    cbase = (cid * E_SC + sid * E_TILE) // C
    pltpu.sync_copy(dst_h.at[pl.ds(cbase, NCH)], didx_all)
    pltpu.sync_copy(batch_h.at[pl.ds((wid * RTW) // C, RTW // C)], bidx_all)
    _fill(buf, C, 16, 1.0)
    plsc.subcore_barrier()

    def step(p, carry):
      for b in range(_NB):
        j = p * _NB + b

        @pl.when(j < NCH)
        def _(j=j, b=b):
          @pl.when(j >= _NB)
          def _():
            pltpu.make_async_copy(
                buf, accd.at[didx_all.at[j - _NB]], ssems[b]).wait()
          pltpu.async_copy(buf, accd.at[didx_all.at[j]], ssems[b], add=True)

      return carry

    lax.fori_loop(0, (NCH + _NB - 1) // _NB, step, 0)
    for kk in range(_NB):
      c = NCH - _NB + kk
      pltpu.make_async_copy(
          buf, accd.at[didx_all.at[c]], ssems[c % _NB]).wait()

    for j in range(RTW // C):
      pltpu.sync_copy(buf, accc.at[bidx_all.at[j]], add=True)

    plsc.subcore_barrier()
    pltpu.sync_copy(accd.at[pl.ds(sid * RTS, RTS)],
                    deg_o.at[cid, pl.ds(sid * RTS, RTS)])

    @pl.when(sid == 0)
    def _():
      pltpu.sync_copy(accc, cnt_o.at[cid])

  return k(dst, batch)


_NB = 4
_LA = 2


def _pipe(g_h, sidx_all, didx_all, rows, gsems, ssems, acc, nch,
          scale_fn=None):
  for b in range(_LA):
    pltpu.async_copy(g_h.at[sidx_all.at[b]], rows.at[b], gsems[b])

  def step(p, carry):
    for b in range(_NB):
      j = p * _NB + b

      @pl.when(j < nch)
      def _(j=j, b=b):
        pltpu.make_async_copy(
            g_h.at[sidx_all.at[j]], rows.at[b], gsems[b]).wait()
        if scale_fn is not None:
          scale_fn(j, b)
        pltpu.async_copy(rows.at[b], acc.at[didx_all.at[j]], ssems[b],
                         add=True)

      jn = j + _LA
      bn = (b + _LA) % _NB

      @pl.when(jnp.logical_and(jn < nch, jn >= _NB))
      def _(jn=jn, bn=bn):
        pltpu.make_async_copy(
            rows.at[bn], acc.at[didx_all.at[jn - _NB]], ssems[bn]).wait()
        pltpu.async_copy(g_h.at[sidx_all.at[jn]], rows.at[bn], gsems[bn])

      @pl.when(jnp.logical_and(jn < nch, jn < _NB))
      def _(jn=jn, bn=bn):
        pltpu.async_copy(g_h.at[sidx_all.at[jn]], rows.at[bn], gsems[bn])

    return carry

  lax.fori_loop(0, (nch + _NB - 1) // _NB, step, 0)
  for kk in range(_NB):
    c = nch - _NB + kk
    pltpu.make_async_copy(
        rows.at[c % _NB], acc.at[didx_all.at[c]], ssems[c % _NB]).wait()


def _spmm_scratch(nch):
  return [
      pltpu.VMEM((nch, C), _i32),
      pltpu.VMEM((nch, C), _i32),
      pltpu.VMEM((_NB, C, D), _f32),
      pltpu.VMEM((C, D), _f32),
  ] + [pltpu.SemaphoreType.DMA] * (2 * _NB) + [
      pltpu.VMEM_SHARED((NP, D), _f32),
  ]


def _sc_spmm_half(g, src2, dst2, ec16=None):
  weighted = ec16 is not None
  scratch = _spmm_scratch(NCH) + (
      [pltpu.VMEM((C, 16), _f32)] if weighted else [])

  @functools.partial(
      pl.kernel, mesh=_MESH, compiler_params=_SC_PARAMS,
      out_type=jax.ShapeDtypeStruct((NC, NP, D), _f32),
      scratch_types=scratch)
  def k(*args):
    if weighted:
      (g_h, src_h, dst_h, ec_h, out_h, sidx_all, didx_all, rows, zbuf,
       *sems, acc, ecv) = args
    else:
      (g_h, src_h, dst_h, out_h, sidx_all, didx_all, rows, zbuf,
       *sems, acc) = args
    gsems, ssems = sems[:_NB], sems[_NB:2 * _NB]
    cid = lax.axis_index("c")
    sid = lax.axis_index("s")
    _fill(zbuf, C, D, 0.0)
    for j in range(RTS // C):
      pltpu.sync_copy(zbuf, acc.at[pl.ds(sid * RTS + j * C, C)])
    cbase = (cid * E_SC + sid * E_TILE) // C
    pltpu.sync_copy(src_h.at[pl.ds(cbase, NCH)], sidx_all)
    pltpu.sync_copy(dst_h.at[pl.ds(cbase, NCH)], didx_all)
    plsc.subcore_barrier()

    if weighted:
      ebase = cid * E_SC + sid * E_TILE

      def scale(j, b):
        pltpu.sync_copy(ec_h.at[pl.ds(ebase + j * C, C)], ecv)

        def srow(r8, c2):
          for u in range(8):
            r = r8 * 8 + u
            ev = ecv[r, :]
            for q in range(D // 16):
              rows[b, r, pl.ds(q * 16, 16)] = (
                  rows[b, r, pl.ds(q * 16, 16)] * ev)
          return c2

        lax.fori_loop(0, C // 8, srow, 0)
    else:
      scale = None

    _pipe(g_h, sidx_all, didx_all, rows, gsems, ssems, acc, NCH, scale)
    plsc.subcore_barrier()
    for j in range(RTS // C):
      r0 = sid * RTS + j * C
      pltpu.sync_copy(acc.at[pl.ds(r0, C)], out_h.at[cid, pl.ds(r0, C)])

  if weighted:
    return k(g, src2, dst2, ec16)
  return k(g, src2, dst2)


def _sc_edge(src2, dst2, p1b, p2b):
  @functools.partial(
      pl.kernel, mesh=_MESH, compiler_params=_SC_PARAMS,
      out_type=(jax.ShapeDtypeStruct((E, 16), _f32),
                jax.ShapeDtypeStruct((NC, NP, 16), _f32)),
      scratch_types=[
          pltpu.VMEM((NCH, C), _i32),
          pltpu.VMEM((NCH, C), _i32),
          pltpu.VMEM((_NB, C, 16), _f32),
          pltpu.VMEM((_NB, C, 16), _f32),
          pltpu.VMEM((_NB, C, 16), _f32),
          pltpu.VMEM((C, 16), _f32),
      ] + [pltpu.SemaphoreType.DMA] * (3 * _NB) + [
          pltpu.VMEM_SHARED((NP, 16), _f32),
      ])
  def k(src_h, dst_h, p1b_h, p2b_h, ec_o, degb_o,
        sidx_all, didx_all, ra, rb, ecv, zbuf, *sems_acc):
    gsems = sems_acc[:_NB]
    ssems = sems_acc[_NB:2 * _NB]
    osems = sems_acc[2 * _NB:3 * _NB]
    acc = sems_acc[3 * _NB]
    cid = lax.axis_index("c")
    sid = lax.axis_index("s")
    _fill(zbuf, C, 16, 0.0)
    for j in range(RTS // C):
      pltpu.sync_copy(zbuf, acc.at[pl.ds(sid * RTS + j * C, C)])
    cbase = (cid * E_SC + sid * E_TILE) // C
    pltpu.sync_copy(src_h.at[pl.ds(cbase, NCH)], sidx_all)
    pltpu.sync_copy(dst_h.at[pl.ds(cbase, NCH)], didx_all)
    plsc.subcore_barrier()

    ebase = cid * E_SC + sid * E_TILE

    def start_gather(j, b):
      pltpu.async_copy(p1b_h.at[sidx_all.at[j]], ra.at[b], gsems[b])
      pltpu.async_copy(p2b_h.at[didx_all.at[j]], rb.at[b], gsems[b])

    for b in range(_LA):
      start_gather(b, b)

    def step(p, carry):
      for b in range(_NB):
        j = p * _NB + b

        @pl.when(j < NCH)
        def _(j=j, b=b):
          pltpu.make_async_copy(
              p1b_h.at[sidx_all.at[j]], ra.at[b], gsems[b]).wait()
          pltpu.make_async_copy(
              p2b_h.at[didx_all.at[j]], rb.at[b], gsems[b]).wait()

          @pl.when(j >= _NB)
          def _():
            pltpu.make_async_copy(
                ecv.at[b], acc.at[didx_all.at[j - _NB]], ssems[b]).wait()
            pltpu.make_async_copy(
                ecv.at[b], ec_o.at[pl.ds(0, C)], osems[b]).wait()

          def erow(r8, c2):
            for u in range(8):
              r = r8 * 8 + u
              z = ra[b, r, :] + rb[b, r, :]
              ecv[b, r, :] = 1.0 / (1.0 + jnp.exp(-z))
            return c2

          lax.fori_loop(0, C // 8, erow, 0)
          pltpu.async_copy(ecv.at[b], acc.at[didx_all.at[j]], ssems[b],
                           add=True)
          pltpu.async_copy(ecv.at[b], ec_o.at[pl.ds(ebase + j * C, C)],
                           osems[b])

        jn = j + _LA
        bn = (b + _LA) % _NB

        @pl.when(jn < NCH)
        def _(jn=jn, bn=bn):
          start_gather(jn, bn)

      return carry

    lax.fori_loop(0, (NCH + _NB - 1) // _NB, step, 0)
    for kk in range(_NB):
      c = NCH - _NB + kk
      pltpu.make_async_copy(
          ecv.at[c % _NB], acc.at[didx_all.at[c]], ssems[c % _NB]).wait()
      pltpu.make_async_copy(
          ecv.at[c % _NB], ec_o.at[pl.ds(0, C)], osems[c % _NB]).wait()
    plsc.subcore_barrier()
    pltpu.sync_copy(acc.at[pl.ds(sid * RTS, RTS)],
                    degb_o.at[cid, pl.ds(sid * RTS, RTS)])

  return k(src2, dst2, p1b, p2b)


def _sc_pool(h, batch):
  @functools.partial(
      pl.kernel, mesh=_MESH, compiler_params=_SC_PARAMS,
      out_type=jax.ShapeDtypeStruct((NC, GP, D), _f32),
      scratch_types=[
          pltpu.VMEM((C,), _i32),
          pltpu.VMEM((C, D), _f32),
          pltpu.VMEM((C, D), _f32),
          pltpu.VMEM_SHARED((GP, D), _f32),
      ])
  def k(h_h, batch_h, out_h, bidx, rows, zbuf, acc):
    cid = lax.axis_index("c")
    sid = lax.axis_index("s")
    wid = cid * NS + sid

    @pl.when(sid == 0)
    def _():
      _fill(zbuf, C, D, 0.0)
      _zero_rows(acc, zbuf, GP)

    plsc.subcore_barrier()
    nbase = wid * RTW

    def step(j, carry):
      off = nbase + j * C
      pltpu.sync_copy(batch_h.at[pl.ds(off, C)], bidx)
      pltpu.sync_copy(h_h.at[pl.ds(off, C)], rows)
      pltpu.sync_copy(rows, acc.at[bidx], add=True)
      return carry

    lax.fori_loop(0, RTW // C, step, 0)
    plsc.subcore_barrier()

    @pl.when(sid == 0)
    def _():
      pltpu.sync_copy(acc, out_h.at[cid])

  return k(h, batch)


_RB = 512
_GRID = NP // _RB


def _rspec(shape3=False, cols=D):
  if shape3:
    return pl.BlockSpec((NC, _RB, cols), lambda j: (0, j, 0))
  return pl.BlockSpec((_RB, cols), lambda j: (j, 0))


def _wspec(r, c):
  return pl.BlockSpec((r, c), lambda j: (0, 0))


def _tc_prep(deg2, x):
  def body(d_r, x_r, q_r, g_r):
    deg = d_r[0, :, 0:1] + d_r[1, :, 0:1] + 1.0
    q = lax.rsqrt(deg)
    q_r[...] = q
    g_r[...] = q * x_r[...]

  return pl.pallas_call(
      body, grid=(_GRID,),
      in_specs=[_rspec(True, 16), _rspec()],
      out_specs=[_rspec(cols=1), _rspec()],
      out_shape=[jax.ShapeDtypeStruct((NP, 1), _f32),
                 jax.ShapeDtypeStruct((NP, D), _f32)])(deg2, x)


def _tc_conv2(S0, g0, q, Wf, bf, Wc, bc):
  def body(s_r, g_r, q_r, wf_r, bf_r, wc_r, bc_r, of_r, oc_r):
    t = q_r[...] * (s_r[0] + s_r[1] + g_r[...])
    hf = jax.nn.relu(jnp.dot(t, wf_r[...], preferred_element_type=_f32)
                     + bf_r[...])
    hc = jax.nn.relu(jnp.dot(t, wc_r[...], preferred_element_type=_f32)
                     + bc_r[...])
    of_r[...] = q_r[...] * hf
    oc_r[...] = q_r[...] * hc

  return pl.pallas_call(
      body, grid=(_GRID,),
      in_specs=[_rspec(True), _rspec(), _rspec(cols=1),
                _wspec(D, D), _wspec(1, D), _wspec(D, D), _wspec(1, D)],
      out_specs=[_rspec(), _rspec()],
      out_shape=[jax.ShapeDtypeStruct((NP, D), _f32),
                 jax.ShapeDtypeStruct((NP, D), _f32)])(
                     S0, g0, q, Wf, bf, Wc, bc)


def _tc_caus(S1f, S1c, g1f, g1c, q, Wf1, bf1, Wc1, bc1, Wsm, bsm):
  def body(sf_r, sc_r, gf_r, gc_r, q_r, wf_r, bf_r, wc_r, bc_r, wsm_r, bsm_r,
           xe_r, nc_r, p1_r, p2_r):
    q = q_r[...]
    xe_r[...] = jax.nn.relu(
        jnp.dot(q * (sf_r[0] + sf_r[1] + gf_r[...]), wf_r[...],
                preferred_element_type=_f32) + bf_r[...])
    h2c = jax.nn.relu(
        jnp.dot(q * (sc_r[0] + sc_r[1] + gc_r[...]), wc_r[...],
                preferred_element_type=_f32) + bc_r[...])
    sm = jnp.dot(h2c, wsm_r[...], preferred_element_type=_f32) + bsm_r[...]
    nc_r[...] = jax.nn.sigmoid(sm[:, 0:1])
    p1_r[...] = jnp.broadcast_to(sm[:, 1:2], (_RB, 16))
    p2_r[...] = jnp.broadcast_to(sm[:, 2:3], (_RB, 16))

  return pl.pallas_call(
      body, grid=(_GRID,),
      in_specs=[_rspec(True), _rspec(True), _rspec(), _rspec(),
                _rspec(cols=1),
                _wspec(D, D), _wspec(1, D), _wspec(D, D), _wspec(1, D),
                _wspec(D, 3), _wspec(1, 3)],
      out_specs=[_rspec(), _rspec(cols=1), _rspec(cols=16), _rspec(cols=16)],
      out_shape=[jax.ShapeDtypeStruct((NP, D), _f32),
                 jax.ShapeDtypeStruct((NP, 1), _f32),
                 jax.ShapeDtypeStruct((NP, 16), _f32),
                 jax.ShapeDtypeStruct((NP, 16), _f32)])(
                     S1f, S1c, g1f, g1c, q, Wf1, bf1, Wc1, bc1, Wsm, bsm)


def _tc_prep2(degb2, xe, nc):
  def body(d_r, xe_r, nc_r, qb_r, gb_r):
    qb = lax.rsqrt(d_r[0, :, 0:1] + d_r[1, :, 0:1] + 1.0)
    qb_r[...] = qb
    gb_r[...] = qb * (xe_r[...] * nc_r[...])

  return pl.pallas_call(
      body, grid=(_GRID,),
      in_specs=[_rspec(True, 16), _rspec(), _rspec(cols=1)],
      out_specs=[_rspec(cols=1), _rspec()],
      out_shape=[jax.ShapeDtypeStruct((NP, 1), _f32),
                 jax.ShapeDtypeStruct((NP, D), _f32)])(degb2, xe, nc)


def _tc_conv1(Sb, gb, qb, nc, W, b):
  def body(s_r, g_r, q_r, nc_r, w_r, b_r, o_r):
    h = jax.nn.relu(
        jnp.dot(q_r[...] * (s_r[0] + s_r[1] + g_r[...]), w_r[...],
                preferred_element_type=_f32) + b_r[...])
    o_r[...] = q_r[...] * (h * nc_r[...])

  return pl.pallas_call(
      body, grid=(_GRID,),
      in_specs=[_rspec(True), _rspec(), _rspec(cols=1), _rspec(cols=1),
                _wspec(D, D), _wspec(1, D)],
      out_specs=_rspec(),
      out_shape=jax.ShapeDtypeStruct((NP, D), _f32))(Sb, gb, qb, nc, W, b)


def _tc_conv_last(Sb, gb, qb, W, b):
  def body(s_r, g_r, q_r, w_r, b_r, o_r):
    o_r[...] = jax.nn.relu(
        jnp.dot(q_r[...] * (s_r[0] + s_r[1] + g_r[...]), w_r[...],
                preferred_element_type=_f32) + b_r[...])

  return pl.pallas_call(
      body, grid=(_GRID,),
      in_specs=[_rspec(True), _rspec(), _rspec(cols=1),
                _wspec(D, D), _wspec(1, D)],
      out_specs=_rspec(),
      out_shape=jax.ShapeDtypeStruct((NP, D), _f32))(Sb, gb, qb, W, b)


def _tc_pred(pooled, cnt2, W_pred, b_pred):
  def body(p_r, c_r, w_r, b_r, o_r):
    sums = p_r[0, :G, :] + p_r[1, :G, :]
    cnt = c_r[0, :G, 0:1] + c_r[1, :G, 0:1]
    hg = sums / jnp.maximum(cnt, 1.0)
    o_r[...] = jnp.dot(hg, w_r[...], preferred_element_type=_f32) + b_r[...]

  return pl.pallas_call(
      body,
      out_shape=jax.ShapeDtypeStruct((G, 10), _f32))(
          pooled, cnt2, W_pred, b_pred)


def kernel(x, edge_index, batch, W_f0, b_f0, W_f1, b_f1, W_c0, b_c0, W_c1,
           b_c1, W_node, b_node, W_edge, b_edge, W_b0, b_b0, W_b1, b_b1,
           W_pred, b_pred):
  src = edge_index[0].astype(_i32)
  dst = edge_index[1].astype(_i32)
  src2 = src.reshape(E // C, C)
  dst2 = dst.reshape(E // C, C)
  xp = jnp.pad(x, ((0, NP - N), (0, 0)))
  batchp = jnp.pad(batch.astype(_i32), (0, NP - N), constant_values=G)
  batch2 = batchp.reshape(NP // C, C)

  deg2, cnt2 = _sc_stats(dst2, batch2)
  q, g0 = _tc_prep(deg2, xp)
  S0 = _sc_spmm_half(g0, src2, dst2)
  g1f, g1c = _tc_conv2(S0, g0, q,
                       W_f0, b_f0.reshape(1, D), W_c0, b_c0.reshape(1, D))
  S1f = _sc_spmm_half(g1f, src2, dst2)
  S1c = _sc_spmm_half(g1c, src2, dst2)
  Wsm = jnp.concatenate([W_node, W_edge[:D], W_edge[D:]], axis=1)
  bsm = jnp.stack([b_node[0], jnp.zeros((), _f32), b_edge[0]]).reshape(1, 3)
  xe, nc, p1b, p2b = _tc_caus(S1f, S1c, g1f, g1c, q,
                              W_f1, b_f1.reshape(1, D), W_c1,
                              b_c1.reshape(1, D), Wsm, bsm)
  ec16, degb2 = _sc_edge(src2, dst2, p1b, p2b)
  qb, gb0 = _tc_prep2(degb2, xe, nc)
  Sb0 = _sc_spmm_half(gb0, src2, dst2, ec16)
  gb1 = _tc_conv1(Sb0, gb0, qb, nc, W_b0, b_b0.reshape(1, D))
  Sb1 = _sc_spmm_half(gb1, src2, dst2, ec16)
  h2 = _tc_conv_last(Sb1, gb1, qb, W_b1, b_b1.reshape(1, D))
  pooled = _sc_pool(h2, batchp)
  return _tc_pred(pooled, cnt2, W_pred, b_pred.reshape(1, 10))


# speedup vs baseline: 21.0223x; 1.1442x over previous
import functools

import jax
import jax.numpy as jnp
from jax import lax
from jax.experimental import pallas as pl
from jax.experimental.pallas import tpu as pltpu
from jax.experimental.pallas import tpu_sc as plsc

N = 10000
NP = 10240
E = 320000
D = 128
G = 64
GP = 128

NC = 2
NS = 16
NW = NC * NS
C = 40

E_SC = E // NC
E_TILE = E_SC // NS
NCH = E_TILE // C
E_TILE_F = E // NS
NCH_F = E_TILE_F // C
RTS = NP // NS
RTW = NP // NW

_MESH = plsc.VectorSubcoreMesh(
    core_axis_name="c", subcore_axis_name="s", num_cores=NC, num_subcores=NS)
_SC_PARAMS = pltpu.CompilerParams(use_tc_tiling_on_sc=False)

_f32 = jnp.float32
_i32 = jnp.int32


def _zero_rows(acc, zbuf, total):
  off = 0
  while off < total:
    n = min(C, total - off)
    pltpu.sync_copy(zbuf.at[pl.ds(0, n)], acc.at[pl.ds(off, n)])
    off += n


def _fill(ref, rows, cols, val, lead=None):
  v = jnp.full((16,), val, _f32)
  cg = cols // 16

  def body(i, carry):
    r = i // cg
    c = i % cg
    if lead is None:
      ref[r, pl.ds(c * 16, 16)] = v
    else:
      ref[lead, r, pl.ds(c * 16, 16)] = v
    return carry

  lax.fori_loop(0, rows * cg, body, 0)


def _sc_stats(dst, batch):
  @functools.partial(
      pl.kernel, mesh=_MESH, compiler_params=_SC_PARAMS,
      out_type=(jax.ShapeDtypeStruct((NC, NP, 16), _f32),
                jax.ShapeDtypeStruct((NC, GP, 16), _f32)),
      scratch_types=[
          pltpu.VMEM((NCH, C), _i32),
          pltpu.VMEM((RTW // C, C), _i32),
          pltpu.VMEM((C, 16), _f32),
      ] + [pltpu.SemaphoreType.DMA] * _NB + [
          pltpu.VMEM_SHARED((NP, 16), _f32),
          pltpu.VMEM_SHARED((GP, 16), _f32),
      ])
  def k(dst_h, batch_h, deg_o, cnt_o, didx_all, bidx_all, buf, *sems_acc):
    ssems = sems_acc[:_NB]
    accd, accc = sems_acc[_NB], sems_acc[_NB + 1]
    cid = lax.axis_index("c")
    sid = lax.axis_index("s")
    wid = cid * NS + sid
    _fill(buf, C, 16, 0.0)
    for j in range(RTS // C):
      pltpu.sync_copy(buf, accd.at[pl.ds(sid * RTS + j * C, C)])

    @pl.when(sid == 0)
    def _():
      _zero_rows(accc, buf, GP)

    cbase = (cid * E_SC + sid * E_TILE) // C
    pltpu.sync_copy(dst_h.at[pl.ds(cbase, NCH)], didx_all)
    pltpu.sync_copy(batch_h.at[pl.ds((wid * RTW) // C, RTW // C)], bidx_all)
    _fill(buf, C, 16, 1.0)
    plsc.subcore_barrier()

    def step(p, carry):
      for b in range(_NB):
        j = p * _NB + b

        @pl.when(j < NCH)
        def _(j=j, b=b):
          @pl.when(j >= _NB)
          def _():
            pltpu.make_async_copy(
                buf, accd.at[didx_all.at[j - _NB]], ssems[b]).wait()
          pltpu.async_copy(buf, accd.at[didx_all.at[j]], ssems[b], add=True)

      return carry

    lax.fori_loop(0, (NCH + _NB - 1) // _NB, step, 0)
    for kk in range(_NB):
      c = NCH - _NB + kk
      pltpu.make_async_copy(
          buf, accd.at[didx_all.at[c]], ssems[c % _NB]).wait()

    for j in range(RTW // C):
      pltpu.sync_copy(buf, accc.at[bidx_all.at[j]], add=True)

    plsc.subcore_barrier()
    pltpu.sync_copy(accd.at[pl.ds(sid * RTS, RTS)],
                    deg_o.at[cid, pl.ds(sid * RTS, RTS)])

    @pl.when(sid == 0)
    def _():
      pltpu.sync_copy(accc, cnt_o.at[cid])

  return k(dst, batch)


_NB = 5
_LA = 2


def _pipe(g_h, sidx_all, didx_all, rows, gsems, ssems, acc, nch,
          scale_fn=None, pre_fn=None):
  for b in range(_LA):
    pltpu.async_copy(g_h.at[sidx_all.at[b]], rows.at[b], gsems[b])
    if pre_fn is not None:
      pre_fn(b, b)

  def step(p, carry):
    for b in range(_NB):
      j = p * _NB + b

      @pl.when(j < nch)
      def _(j=j, b=b):
        pltpu.make_async_copy(
            g_h.at[sidx_all.at[j]], rows.at[b], gsems[b]).wait()
        if scale_fn is not None:
          scale_fn(j, b)
        pltpu.async_copy(rows.at[b], acc.at[didx_all.at[j]], ssems[b],
                         add=True)

      jn = j + _LA
      bn = (b + _LA) % _NB

      @pl.when(jnp.logical_and(jn < nch, jn >= _NB))
      def _(jn=jn, bn=bn):
        pltpu.make_async_copy(
            rows.at[bn], acc.at[didx_all.at[jn - _NB]], ssems[bn]).wait()
        pltpu.async_copy(g_h.at[sidx_all.at[jn]], rows.at[bn], gsems[bn])
        if pre_fn is not None:
          pre_fn(jn, bn)

      @pl.when(jnp.logical_and(jn < nch, jn < _NB))
      def _(jn=jn, bn=bn):
        pltpu.async_copy(g_h.at[sidx_all.at[jn]], rows.at[bn], gsems[bn])
        if pre_fn is not None:
          pre_fn(jn, bn)

    return carry

  lax.fori_loop(0, (nch + _NB - 1) // _NB, step, 0)
  for kk in range(_NB):
    c = nch - _NB + kk
    pltpu.make_async_copy(
        rows.at[c % _NB], acc.at[didx_all.at[c]], ssems[c % _NB]).wait()


def _spmm_scratch(nch):
  return [
      pltpu.VMEM((nch, C), _i32),
      pltpu.VMEM((nch, C), _i32),
      pltpu.VMEM((_NB, C, D), _f32),
  ] + [pltpu.SemaphoreType.DMA] * (2 * _NB) + [
      pltpu.VMEM_SHARED((NP, D), _f32),
  ]


def _sc_spmm_half(g, src2, dst2, ec16=None):
  weighted = ec16 is not None
  scratch = _spmm_scratch(NCH) + (
      [pltpu.VMEM((_NB, C, 16), _f32)] +
      [pltpu.SemaphoreType.DMA] * _NB if weighted else [])

  @functools.partial(
      pl.kernel, mesh=_MESH, compiler_params=_SC_PARAMS,
      out_type=jax.ShapeDtypeStruct((NC, NP, D), _f32),
      scratch_types=scratch)
  def k(*args):
    if weighted:
      (g_h, src_h, dst_h, ec_h, out_h, sidx_all, didx_all, rows,
       *rest) = args
      ecv = rest[2 * _NB + 1]
      esems = rest[2 * _NB + 2:]
    else:
      (g_h, src_h, dst_h, out_h, sidx_all, didx_all, rows, *rest) = args
    gsems, ssems = rest[:_NB], rest[_NB:2 * _NB]
    acc = rest[2 * _NB]
    cid = lax.axis_index("c")
    sid = lax.axis_index("s")
    _fill(rows, C, D, 0.0, lead=0)
    for j in range(RTS // C):
      pltpu.sync_copy(rows.at[0], acc.at[pl.ds(sid * RTS + j * C, C)])
    cbase = (cid * E_SC + sid * E_TILE) // C
    pltpu.sync_copy(src_h.at[pl.ds(cbase, NCH)], sidx_all)
    pltpu.sync_copy(dst_h.at[pl.ds(cbase, NCH)], didx_all)
    plsc.subcore_barrier()

    if weighted:
      ebase = cid * E_SC + sid * E_TILE

      def pre(jn, bn):
        pltpu.async_copy(ec_h.at[pl.ds(ebase + jn * C, C)], ecv.at[bn],
                         esems[bn])

      def scale(j, b):
        pltpu.make_async_copy(
            ec_h.at[pl.ds(0, C)], ecv.at[b], esems[b]).wait()

        def srow(r8, c2):
          for u in range(8):
            r = r8 * 8 + u
            ev = ecv[b, r, :]
            for q in range(D // 16):
              rows[b, r, pl.ds(q * 16, 16)] = (
                  rows[b, r, pl.ds(q * 16, 16)] * ev)
          return c2

        lax.fori_loop(0, C // 8, srow, 0)
    else:
      scale = None
      pre = None

    _pipe(g_h, sidx_all, didx_all, rows, gsems, ssems, acc, NCH, scale, pre)
    plsc.subcore_barrier()
    for j in range(RTS // C):
      r0 = sid * RTS + j * C
      pltpu.sync_copy(acc.at[pl.ds(r0, C)], out_h.at[cid, pl.ds(r0, C)])

  if weighted:
    return k(g, src2, dst2, ec16)
  return k(g, src2, dst2)


def _sc_edge(src2, dst2, p1b, p2b):
  @functools.partial(
      pl.kernel, mesh=_MESH, compiler_params=_SC_PARAMS,
      out_type=(jax.ShapeDtypeStruct((E, 16), _f32),
                jax.ShapeDtypeStruct((NC, NP, 16), _f32)),
      scratch_types=[
          pltpu.VMEM((NCH, C), _i32),
          pltpu.VMEM((NCH, C), _i32),
          pltpu.VMEM((_NB, C, 16), _f32),
          pltpu.VMEM((_NB, C, 16), _f32),
          pltpu.VMEM((_NB, C, 16), _f32),
          pltpu.VMEM((C, 16), _f32),
      ] + [pltpu.SemaphoreType.DMA] * (3 * _NB) + [
          pltpu.VMEM_SHARED((NP, 16), _f32),
      ])
  def k(src_h, dst_h, p1b_h, p2b_h, ec_o, degb_o,
        sidx_all, didx_all, ra, rb, ecv, zbuf, *sems_acc):
    gsems = sems_acc[:_NB]
    ssems = sems_acc[_NB:2 * _NB]
    osems = sems_acc[2 * _NB:3 * _NB]
    acc = sems_acc[3 * _NB]
    cid = lax.axis_index("c")
    sid = lax.axis_index("s")
    _fill(zbuf, C, 16, 0.0)
    for j in range(RTS // C):
      pltpu.sync_copy(zbuf, acc.at[pl.ds(sid * RTS + j * C, C)])
    cbase = (cid * E_SC + sid * E_TILE) // C
    pltpu.sync_copy(src_h.at[pl.ds(cbase, NCH)], sidx_all)
    pltpu.sync_copy(dst_h.at[pl.ds(cbase, NCH)], didx_all)
    plsc.subcore_barrier()

    ebase = cid * E_SC + sid * E_TILE

    def start_gather(j, b):
      pltpu.async_copy(p1b_h.at[sidx_all.at[j]], ra.at[b], gsems[b])
      pltpu.async_copy(p2b_h.at[didx_all.at[j]], rb.at[b], gsems[b])

    for b in range(_LA):
      start_gather(b, b)

    def step(p, carry):
      for b in range(_NB):
        j = p * _NB + b

        @pl.when(j < NCH)
        def _(j=j, b=b):
          pltpu.make_async_copy(
              p1b_h.at[sidx_all.at[j]], ra.at[b], gsems[b]).wait()
          pltpu.make_async_copy(
              p2b_h.at[didx_all.at[j]], rb.at[b], gsems[b]).wait()

          @pl.when(j >= _NB)
          def _():
            pltpu.make_async_copy(
                ecv.at[b], acc.at[didx_all.at[j - _NB]], ssems[b]).wait()
            pltpu.make_async_copy(
                ecv.at[b], ec_o.at[pl.ds(0, C)], osems[b]).wait()

          def erow(r8, c2):
            for u in range(8):
              r = r8 * 8 + u
              z = ra[b, r, :] + rb[b, r, :]
              ecv[b, r, :] = 1.0 / (1.0 + jnp.exp(-z))
            return c2

          lax.fori_loop(0, C // 8, erow, 0)
          pltpu.async_copy(ecv.at[b], acc.at[didx_all.at[j]], ssems[b],
                           add=True)
          pltpu.async_copy(ecv.at[b], ec_o.at[pl.ds(ebase + j * C, C)],
                           osems[b])

        jn = j + _LA
        bn = (b + _LA) % _NB

        @pl.when(jn < NCH)
        def _(jn=jn, bn=bn):
          start_gather(jn, bn)

      return carry

    lax.fori_loop(0, (NCH + _NB - 1) // _NB, step, 0)
    for kk in range(_NB):
      c = NCH - _NB + kk
      pltpu.make_async_copy(
          ecv.at[c % _NB], acc.at[didx_all.at[c]], ssems[c % _NB]).wait()
      pltpu.make_async_copy(
          ecv.at[c % _NB], ec_o.at[pl.ds(0, C)], osems[c % _NB]).wait()
    plsc.subcore_barrier()
    pltpu.sync_copy(acc.at[pl.ds(sid * RTS, RTS)],
                    degb_o.at[cid, pl.ds(sid * RTS, RTS)])

  return k(src2, dst2, p1b, p2b)


def _sc_pool(h, batch):
  @functools.partial(
      pl.kernel, mesh=_MESH, compiler_params=_SC_PARAMS,
      out_type=jax.ShapeDtypeStruct((NC, GP, D), _f32),
      scratch_types=[
          pltpu.VMEM((C,), _i32),
          pltpu.VMEM((C, D), _f32),
          pltpu.VMEM((C, D), _f32),
          pltpu.VMEM_SHARED((GP, D), _f32),
      ])
  def k(h_h, batch_h, out_h, bidx, rows, zbuf, acc):
    cid = lax.axis_index("c")
    sid = lax.axis_index("s")
    wid = cid * NS + sid

    @pl.when(sid == 0)
    def _():
      _fill(zbuf, C, D, 0.0)
      _zero_rows(acc, zbuf, GP)

    plsc.subcore_barrier()
    nbase = wid * RTW

    def step(j, carry):
      off = nbase + j * C
      pltpu.sync_copy(batch_h.at[pl.ds(off, C)], bidx)
      pltpu.sync_copy(h_h.at[pl.ds(off, C)], rows)
      pltpu.sync_copy(rows, acc.at[bidx], add=True)
      return carry

    lax.fori_loop(0, RTW // C, step, 0)
    plsc.subcore_barrier()

    @pl.when(sid == 0)
    def _():
      pltpu.sync_copy(acc, out_h.at[cid])

  return k(h, batch)


_RB = 512
_GRID = NP // _RB


def _rspec(shape3=False, cols=D):
  if shape3:
    return pl.BlockSpec((NC, _RB, cols), lambda j: (0, j, 0))
  return pl.BlockSpec((_RB, cols), lambda j: (j, 0))


def _wspec(r, c):
  return pl.BlockSpec((r, c), lambda j: (0, 0))


def _tc_prep(deg2, x):
  def body(d_r, x_r, q_r, g_r):
    deg = d_r[0, :, 0:1] + d_r[1, :, 0:1] + 1.0
    q = lax.rsqrt(deg)
    q_r[...] = q
    g_r[...] = q * x_r[...]

  return pl.pallas_call(
      body, grid=(_GRID,),
      in_specs=[_rspec(True, 16), _rspec()],
      out_specs=[_rspec(cols=1), _rspec()],
      out_shape=[jax.ShapeDtypeStruct((NP, 1), _f32),
                 jax.ShapeDtypeStruct((NP, D), _f32)])(deg2, x)


def _tc_conv2(S0, g0, q, Wf, bf, Wc, bc):
  def body(s_r, g_r, q_r, wf_r, bf_r, wc_r, bc_r, of_r, oc_r):
    t = q_r[...] * (s_r[0] + s_r[1] + g_r[...])
    hf = jax.nn.relu(jnp.dot(t, wf_r[...], preferred_element_type=_f32)
                     + bf_r[...])
    hc = jax.nn.relu(jnp.dot(t, wc_r[...], preferred_element_type=_f32)
                     + bc_r[...])
    of_r[...] = q_r[...] * hf
    oc_r[...] = q_r[...] * hc

  return pl.pallas_call(
      body, grid=(_GRID,),
      in_specs=[_rspec(True), _rspec(), _rspec(cols=1),
                _wspec(D, D), _wspec(1, D), _wspec(D, D), _wspec(1, D)],
      out_specs=[_rspec(), _rspec()],
      out_shape=[jax.ShapeDtypeStruct((NP, D), _f32),
                 jax.ShapeDtypeStruct((NP, D), _f32)])(
                     S0, g0, q, Wf, bf, Wc, bc)


def _tc_caus(S1f, S1c, g1f, g1c, q, Wf1, bf1, Wc1, bc1, Wsm, bsm):
  def body(sf_r, sc_r, gf_r, gc_r, q_r, wf_r, bf_r, wc_r, bc_r, wsm_r, bsm_r,
           xe_r, nc_r, p1_r, p2_r):
    q = q_r[...]
    xe_r[...] = jax.nn.relu(
        jnp.dot(q * (sf_r[0] + sf_r[1] + gf_r[...]), wf_r[...],
                preferred_element_type=_f32) + bf_r[...])
    h2c = jax.nn.relu(
        jnp.dot(q * (sc_r[0] + sc_r[1] + gc_r[...]), wc_r[...],
                preferred_element_type=_f32) + bc_r[...])
    sm = jnp.dot(h2c, wsm_r[...], preferred_element_type=_f32) + bsm_r[...]
    nc_r[...] = jax.nn.sigmoid(sm[:, 0:1])
    p1_r[...] = jnp.broadcast_to(sm[:, 1:2], (_RB, 16))
    p2_r[...] = jnp.broadcast_to(sm[:, 2:3], (_RB, 16))

  return pl.pallas_call(
      body, grid=(_GRID,),
      in_specs=[_rspec(True), _rspec(True), _rspec(), _rspec(),
                _rspec(cols=1),
                _wspec(D, D), _wspec(1, D), _wspec(D, D), _wspec(1, D),
                _wspec(D, 3), _wspec(1, 3)],
      out_specs=[_rspec(), _rspec(cols=1), _rspec(cols=16), _rspec(cols=16)],
      out_shape=[jax.ShapeDtypeStruct((NP, D), _f32),
                 jax.ShapeDtypeStruct((NP, 1), _f32),
                 jax.ShapeDtypeStruct((NP, 16), _f32),
                 jax.ShapeDtypeStruct((NP, 16), _f32)])(
                     S1f, S1c, g1f, g1c, q, Wf1, bf1, Wc1, bc1, Wsm, bsm)


def _tc_prep2(degb2, xe, nc):
  def body(d_r, xe_r, nc_r, qb_r, gb_r):
    qb = lax.rsqrt(d_r[0, :, 0:1] + d_r[1, :, 0:1] + 1.0)
    qb_r[...] = qb
    gb_r[...] = qb * (xe_r[...] * nc_r[...])

  return pl.pallas_call(
      body, grid=(_GRID,),
      in_specs=[_rspec(True, 16), _rspec(), _rspec(cols=1)],
      out_specs=[_rspec(cols=1), _rspec()],
      out_shape=[jax.ShapeDtypeStruct((NP, 1), _f32),
                 jax.ShapeDtypeStruct((NP, D), _f32)])(degb2, xe, nc)


def _tc_conv1(Sb, gb, qb, nc, W, b):
  def body(s_r, g_r, q_r, nc_r, w_r, b_r, o_r):
    h = jax.nn.relu(
        jnp.dot(q_r[...] * (s_r[0] + s_r[1] + g_r[...]), w_r[...],
                preferred_element_type=_f32) + b_r[...])
    o_r[...] = q_r[...] * (h * nc_r[...])

  return pl.pallas_call(
      body, grid=(_GRID,),
      in_specs=[_rspec(True), _rspec(), _rspec(cols=1), _rspec(cols=1),
                _wspec(D, D), _wspec(1, D)],
      out_specs=_rspec(),
      out_shape=jax.ShapeDtypeStruct((NP, D), _f32))(Sb, gb, qb, nc, W, b)


def _tc_conv_last(Sb, gb, qb, W, b):
  def body(s_r, g_r, q_r, w_r, b_r, o_r):
    o_r[...] = jax.nn.relu(
        jnp.dot(q_r[...] * (s_r[0] + s_r[1] + g_r[...]), w_r[...],
                preferred_element_type=_f32) + b_r[...])

  return pl.pallas_call(
      body, grid=(_GRID,),
      in_specs=[_rspec(True), _rspec(), _rspec(cols=1),
                _wspec(D, D), _wspec(1, D)],
      out_specs=_rspec(),
      out_shape=jax.ShapeDtypeStruct((NP, D), _f32))(Sb, gb, qb, W, b)


def _tc_pred(pooled, cnt2, W_pred, b_pred):
  def body(p_r, c_r, w_r, b_r, o_r):
    sums = p_r[0, :G, :] + p_r[1, :G, :]
    cnt = c_r[0, :G, 0:1] + c_r[1, :G, 0:1]
    hg = sums / jnp.maximum(cnt, 1.0)
    o_r[...] = jnp.dot(hg, w_r[...], preferred_element_type=_f32) + b_r[...]

  return pl.pallas_call(
      body,
      out_shape=jax.ShapeDtypeStruct((G, 10), _f32))(
          pooled, cnt2, W_pred, b_pred)


def kernel(x, edge_index, batch, W_f0, b_f0, W_f1, b_f1, W_c0, b_c0, W_c1,
           b_c1, W_node, b_node, W_edge, b_edge, W_b0, b_b0, W_b1, b_b1,
           W_pred, b_pred):
  src = edge_index[0].astype(_i32)
  dst = edge_index[1].astype(_i32)
  src2 = src.reshape(E // C, C)
  dst2 = dst.reshape(E // C, C)
  xp = jnp.pad(x, ((0, NP - N), (0, 0)))
  batchp = jnp.pad(batch.astype(_i32), (0, NP - N), constant_values=G)
  batch2 = batchp.reshape(NP // C, C)

  deg2, cnt2 = _sc_stats(dst2, batch2)
  q, g0 = _tc_prep(deg2, xp)
  S0 = _sc_spmm_half(g0, src2, dst2)
  g1f, g1c = _tc_conv2(S0, g0, q,
                       W_f0, b_f0.reshape(1, D), W_c0, b_c0.reshape(1, D))
  S1f = _sc_spmm_half(g1f, src2, dst2)
  S1c = _sc_spmm_half(g1c, src2, dst2)
  Wsm = jnp.concatenate([W_node, W_edge[:D], W_edge[D:]], axis=1)
  bsm = jnp.stack([b_node[0], jnp.zeros((), _f32), b_edge[0]]).reshape(1, 3)
  xe, nc, p1b, p2b = _tc_caus(S1f, S1c, g1f, g1c, q,
                              W_f1, b_f1.reshape(1, D), W_c1,
                              b_c1.reshape(1, D), Wsm, bsm)
  ec16, degb2 = _sc_edge(src2, dst2, p1b, p2b)
  qb, gb0 = _tc_prep2(degb2, xe, nc)
  Sb0 = _sc_spmm_half(gb0, src2, dst2, ec16)
  gb1 = _tc_conv1(Sb0, gb0, qb, nc, W_b0, b_b0.reshape(1, D))
  Sb1 = _sc_spmm_half(gb1, src2, dst2, ec16)
  h2 = _tc_conv_last(Sb1, gb1, qb, W_b1, b_b1.reshape(1, D))
  pooled = _sc_pool(h2, batchp)
  return _tc_pred(pooled, cnt2, W_pred, b_pred.reshape(1, 10))


# speedup vs baseline: 25.8439x; 1.2294x over previous
import functools

import jax
import jax.numpy as jnp
from jax import lax
from jax.experimental import pallas as pl
from jax.experimental.pallas import tpu as pltpu
from jax.experimental.pallas import tpu_sc as plsc

N = 10000
NP = 10240
E = 320000
D = 128
G = 64
GP = 128

NC = 2
NS = 16
NW = NC * NS
C = 40

E_SC = E // NC
E_TILE = E_SC // NS
NCH = E_TILE // C
E_TILE_F = E // NS
NCH_F = E_TILE_F // C
RTS = NP // NS
RTW = NP // NW

_MESH = plsc.VectorSubcoreMesh(
    core_axis_name="c", subcore_axis_name="s", num_cores=NC, num_subcores=NS)
_SC_PARAMS = pltpu.CompilerParams(use_tc_tiling_on_sc=False)

_f32 = jnp.float32
_i32 = jnp.int32


def _zero_rows(acc, zbuf, total):
  off = 0
  while off < total:
    n = min(C, total - off)
    pltpu.sync_copy(zbuf.at[pl.ds(0, n)], acc.at[pl.ds(off, n)])
    off += n


def _fill(ref, rows, cols, val, lead=None):
  v = jnp.full((16,), val, _f32)
  cg = cols // 16

  def body(i, carry):
    r = i // cg
    c = i % cg
    if lead is None:
      ref[r, pl.ds(c * 16, 16)] = v
    else:
      ref[lead, r, pl.ds(c * 16, 16)] = v
    return carry

  lax.fori_loop(0, rows * cg, body, 0)


def _sc_stats(dst, batch):
  @functools.partial(
      pl.kernel, mesh=_MESH, compiler_params=_SC_PARAMS,
      out_type=(jax.ShapeDtypeStruct((NC, NP, 16), _f32),
                jax.ShapeDtypeStruct((NC, GP, 16), _f32)),
      scratch_types=[
          pltpu.VMEM((NCH, C), _i32),
          pltpu.VMEM((RTW // C, C), _i32),
          pltpu.VMEM((C, 16), _f32),
      ] + [pltpu.SemaphoreType.DMA] * _NB + [
          pltpu.VMEM_SHARED((NP, 16), _f32),
          pltpu.VMEM_SHARED((GP, 16), _f32),
      ])
  def k(dst_h, batch_h, deg_o, cnt_o, didx_all, bidx_all, buf, *sems_acc):
    ssems = sems_acc[:_NB]
    accd, accc = sems_acc[_NB], sems_acc[_NB + 1]
    cid = lax.axis_index("c")
    sid = lax.axis_index("s")
    wid = cid * NS + sid
    _fill(buf, C, 16, 0.0)
    for j in range(RTS // C):
      pltpu.sync_copy(buf, accd.at[pl.ds(sid * RTS + j * C, C)])

    @pl.when(sid == 0)
    def _():
      _zero_rows(accc, buf, GP)

    cbase = (cid * E_SC + sid * E_TILE) // C
    pltpu.sync_copy(dst_h.at[pl.ds(cbase, NCH)], didx_all)
    pltpu.sync_copy(batch_h.at[pl.ds((wid * RTW) // C, RTW // C)], bidx_all)
    _fill(buf, C, 16, 1.0)
    plsc.subcore_barrier()

    def step(p, carry):
      for b in range(_NB):
        j = p * _NB + b

        @pl.when(j < NCH)
        def _(j=j, b=b):
          @pl.when(j >= _NB)
          def _():
            pltpu.make_async_copy(
                buf, accd.at[didx_all.at[j - _NB]], ssems[b]).wait()
          pltpu.async_copy(buf, accd.at[didx_all.at[j]], ssems[b], add=True)

      return carry

    lax.fori_loop(0, (NCH + _NB - 1) // _NB, step, 0)
    for kk in range(_NB):
      c = NCH - _NB + kk
      pltpu.make_async_copy(
          buf, accd.at[didx_all.at[c]], ssems[c % _NB]).wait()

    for j in range(RTW // C):
      pltpu.sync_copy(buf, accc.at[bidx_all.at[j]], add=True)

    plsc.subcore_barrier()
    pltpu.sync_copy(accd.at[pl.ds(sid * RTS, RTS)],
                    deg_o.at[cid, pl.ds(sid * RTS, RTS)])

    @pl.when(sid == 0)
    def _():
      pltpu.sync_copy(accc, cnt_o.at[cid])

  return k(dst, batch)


_NB = 5
_LA = 3


def _pipe(g_h, sidx_all, didx_all, rows, gsems, ssems, acc, nch,
          scale_fn=None, pre_fn=None):
  for b in range(_LA):
    pltpu.async_copy(g_h.at[sidx_all.at[b]], rows.at[b], gsems[b])
    if pre_fn is not None:
      pre_fn(b, b)

  def step(p, carry):
    for b in range(_NB):
      j = p * _NB + b

      @pl.when(j < nch)
      def _(j=j, b=b):
        pltpu.make_async_copy(
            g_h.at[sidx_all.at[j]], rows.at[b], gsems[b]).wait()
        if scale_fn is not None:
          scale_fn(j, b)
        pltpu.async_copy(rows.at[b], acc.at[didx_all.at[j]], ssems[b],
                         add=True)

      jn = j + _LA
      bn = (b + _LA) % _NB

      @pl.when(jnp.logical_and(jn < nch, jn >= _NB))
      def _(jn=jn, bn=bn):
        pltpu.make_async_copy(
            rows.at[bn], acc.at[didx_all.at[jn - _NB]], ssems[bn]).wait()
        pltpu.async_copy(g_h.at[sidx_all.at[jn]], rows.at[bn], gsems[bn])
        if pre_fn is not None:
          pre_fn(jn, bn)

      @pl.when(jnp.logical_and(jn < nch, jn < _NB))
      def _(jn=jn, bn=bn):
        pltpu.async_copy(g_h.at[sidx_all.at[jn]], rows.at[bn], gsems[bn])
        if pre_fn is not None:
          pre_fn(jn, bn)

    return carry

  lax.fori_loop(0, (nch + _NB - 1) // _NB, step, 0)
  for kk in range(_NB):
    c = nch - _NB + kk
    pltpu.make_async_copy(
        rows.at[c % _NB], acc.at[didx_all.at[c]], ssems[c % _NB]).wait()


def _spmm_scratch(nch):
  return [
      pltpu.VMEM((nch, C), _i32),
      pltpu.VMEM((nch, C), _i32),
      pltpu.VMEM((_NB, C, D), _f32),
  ] + [pltpu.SemaphoreType.DMA] * (2 * _NB) + [
      pltpu.VMEM_SHARED((NP, D), _f32),
  ]


def _sc_spmm_half(g, src2, dst2, ec16=None):
  weighted = ec16 is not None
  scratch = _spmm_scratch(NCH) + (
      [pltpu.VMEM((_NB, C, 16), _f32)] +
      [pltpu.SemaphoreType.DMA] * _NB if weighted else [])

  @functools.partial(
      pl.kernel, mesh=_MESH, compiler_params=_SC_PARAMS,
      out_type=jax.ShapeDtypeStruct((NC, NP, D), _f32),
      scratch_types=scratch)
  def k(*args):
    if weighted:
      (g_h, src_h, dst_h, ec_h, out_h, sidx_all, didx_all, rows,
       *rest) = args
      ecv = rest[2 * _NB + 1]
      esems = rest[2 * _NB + 2:]
    else:
      (g_h, src_h, dst_h, out_h, sidx_all, didx_all, rows, *rest) = args
    gsems, ssems = rest[:_NB], rest[_NB:2 * _NB]
    acc = rest[2 * _NB]
    cid = lax.axis_index("c")
    sid = lax.axis_index("s")
    _fill(rows, C, D, 0.0, lead=0)
    for j in range(RTS // C):
      pltpu.sync_copy(rows.at[0], acc.at[pl.ds(sid * RTS + j * C, C)])
    cbase = (cid * E_SC + sid * E_TILE) // C
    pltpu.sync_copy(src_h.at[pl.ds(cbase, NCH)], sidx_all)
    pltpu.sync_copy(dst_h.at[pl.ds(cbase, NCH)], didx_all)
    plsc.subcore_barrier()

    if weighted:
      ebase = cid * E_SC + sid * E_TILE

      def pre(jn, bn):
        pltpu.async_copy(ec_h.at[pl.ds(ebase + jn * C, C)], ecv.at[bn],
                         esems[bn])

      def scale(j, b):
        pltpu.make_async_copy(
            ec_h.at[pl.ds(0, C)], ecv.at[b], esems[b]).wait()

        def srow(r8, c2):
          for u in range(8):
            r = r8 * 8 + u
            ev = ecv[b, r, :]
            for q in range(D // 16):
              rows[b, r, pl.ds(q * 16, 16)] = (
                  rows[b, r, pl.ds(q * 16, 16)] * ev)
          return c2

        lax.fori_loop(0, C // 8, srow, 0)
    else:
      scale = None
      pre = None

    _pipe(g_h, sidx_all, didx_all, rows, gsems, ssems, acc, NCH, scale, pre)
    plsc.subcore_barrier()
    for j in range(RTS // C):
      r0 = sid * RTS + j * C
      pltpu.sync_copy(acc.at[pl.ds(r0, C)], out_h.at[cid, pl.ds(r0, C)])

  if weighted:
    return k(g, src2, dst2, ec16)
  return k(g, src2, dst2)


def _sc_edge(src2, dst2, p1b, p2b):
  @functools.partial(
      pl.kernel, mesh=_MESH, compiler_params=_SC_PARAMS,
      out_type=(jax.ShapeDtypeStruct((E, 16), _f32),
                jax.ShapeDtypeStruct((NC, NP, 16), _f32)),
      scratch_types=[
          pltpu.VMEM((NCH, C), _i32),
          pltpu.VMEM((NCH, C), _i32),
          pltpu.VMEM((_NB, C, 16), _f32),
          pltpu.VMEM((_NB, C, 16), _f32),
          pltpu.VMEM((_NB, C, 16), _f32),
          pltpu.VMEM((C, 16), _f32),
      ] + [pltpu.SemaphoreType.DMA] * (3 * _NB) + [
          pltpu.VMEM_SHARED((NP, 16), _f32),
      ])
  def k(src_h, dst_h, p1b_h, p2b_h, ec_o, degb_o,
        sidx_all, didx_all, ra, rb, ecv, zbuf, *sems_acc):
    gsems = sems_acc[:_NB]
    ssems = sems_acc[_NB:2 * _NB]
    osems = sems_acc[2 * _NB:3 * _NB]
    acc = sems_acc[3 * _NB]
    cid = lax.axis_index("c")
    sid = lax.axis_index("s")
    _fill(zbuf, C, 16, 0.0)
    for j in range(RTS // C):
      pltpu.sync_copy(zbuf, acc.at[pl.ds(sid * RTS + j * C, C)])
    cbase = (cid * E_SC + sid * E_TILE) // C
    pltpu.sync_copy(src_h.at[pl.ds(cbase, NCH)], sidx_all)
    pltpu.sync_copy(dst_h.at[pl.ds(cbase, NCH)], didx_all)
    plsc.subcore_barrier()

    ebase = cid * E_SC + sid * E_TILE

    def start_gather(j, b):
      pltpu.async_copy(p1b_h.at[sidx_all.at[j]], ra.at[b], gsems[b])
      pltpu.async_copy(p2b_h.at[didx_all.at[j]], rb.at[b], gsems[b])

    for b in range(_LA):
      start_gather(b, b)

    def step(p, carry):
      for b in range(_NB):
        j = p * _NB + b

        @pl.when(j < NCH)
        def _(j=j, b=b):
          pltpu.make_async_copy(
              p1b_h.at[sidx_all.at[j]], ra.at[b], gsems[b]).wait()
          pltpu.make_async_copy(
              p2b_h.at[didx_all.at[j]], rb.at[b], gsems[b]).wait()

          @pl.when(j >= _NB)
          def _():
            pltpu.make_async_copy(
                ecv.at[b], acc.at[didx_all.at[j - _NB]], ssems[b]).wait()
            pltpu.make_async_copy(
                ecv.at[b], ec_o.at[pl.ds(0, C)], osems[b]).wait()

          def erow(r8, c2):
            for u in range(8):
              r = r8 * 8 + u
              z = ra[b, r, :] + rb[b, r, :]
              ecv[b, r, :] = 1.0 / (1.0 + jnp.exp(-z))
            return c2

          lax.fori_loop(0, C // 8, erow, 0)
          pltpu.async_copy(ecv.at[b], acc.at[didx_all.at[j]], ssems[b],
                           add=True)
          pltpu.async_copy(ecv.at[b], ec_o.at[pl.ds(ebase + j * C, C)],
                           osems[b])

        jn = j + _LA
        bn = (b + _LA) % _NB

        @pl.when(jn < NCH)
        def _(jn=jn, bn=bn):
          start_gather(jn, bn)

      return carry

    lax.fori_loop(0, (NCH + _NB - 1) // _NB, step, 0)
    for kk in range(_NB):
      c = NCH - _NB + kk
      pltpu.make_async_copy(
          ecv.at[c % _NB], acc.at[didx_all.at[c]], ssems[c % _NB]).wait()
      pltpu.make_async_copy(
          ecv.at[c % _NB], ec_o.at[pl.ds(0, C)], osems[c % _NB]).wait()
    plsc.subcore_barrier()
    pltpu.sync_copy(acc.at[pl.ds(sid * RTS, RTS)],
                    degb_o.at[cid, pl.ds(sid * RTS, RTS)])

  return k(src2, dst2, p1b, p2b)


def _sc_pool(h, batch):
  @functools.partial(
      pl.kernel, mesh=_MESH, compiler_params=_SC_PARAMS,
      out_type=jax.ShapeDtypeStruct((NC, GP, D), _f32),
      scratch_types=[
          pltpu.VMEM((C,), _i32),
          pltpu.VMEM((C, D), _f32),
          pltpu.VMEM((C, D), _f32),
          pltpu.VMEM_SHARED((GP, D), _f32),
      ])
  def k(h_h, batch_h, out_h, bidx, rows, zbuf, acc):
    cid = lax.axis_index("c")
    sid = lax.axis_index("s")
    wid = cid * NS + sid

    @pl.when(sid == 0)
    def _():
      _fill(zbuf, C, D, 0.0)
      _zero_rows(acc, zbuf, GP)

    plsc.subcore_barrier()
    nbase = wid * RTW

    def step(j, carry):
      off = nbase + j * C
      pltpu.sync_copy(batch_h.at[pl.ds(off, C)], bidx)
      pltpu.sync_copy(h_h.at[pl.ds(off, C)], rows)
      pltpu.sync_copy(rows, acc.at[bidx], add=True)
      return carry

    lax.fori_loop(0, RTW // C, step, 0)
    plsc.subcore_barrier()

    @pl.when(sid == 0)
    def _():
      pltpu.sync_copy(acc, out_h.at[cid])

  return k(h, batch)


_RB = 512
_GRID = NP // _RB


def _rspec(shape3=False, cols=D):
  if shape3:
    return pl.BlockSpec((NC, _RB, cols), lambda j: (0, j, 0))
  return pl.BlockSpec((_RB, cols), lambda j: (j, 0))


def _wspec(r, c):
  return pl.BlockSpec((r, c), lambda j: (0, 0))


def _tc_prep(deg2, x):
  def body(d_r, x_r, q_r, g_r):
    deg = d_r[0, :, 0:1] + d_r[1, :, 0:1] + 1.0
    q = lax.rsqrt(deg)
    q_r[...] = q
    g_r[...] = q * x_r[...]

  return pl.pallas_call(
      body, grid=(_GRID,),
      in_specs=[_rspec(True, 16), _rspec()],
      out_specs=[_rspec(cols=1), _rspec()],
      out_shape=[jax.ShapeDtypeStruct((NP, 1), _f32),
                 jax.ShapeDtypeStruct((NP, D), _f32)])(deg2, x)


def _tc_conv2(S0, g0, q, Wf, bf, Wc, bc):
  def body(s_r, g_r, q_r, wf_r, bf_r, wc_r, bc_r, of_r, oc_r):
    t = q_r[...] * (s_r[0] + s_r[1] + g_r[...])
    hf = jax.nn.relu(jnp.dot(t, wf_r[...], preferred_element_type=_f32)
                     + bf_r[...])
    hc = jax.nn.relu(jnp.dot(t, wc_r[...], preferred_element_type=_f32)
                     + bc_r[...])
    of_r[...] = q_r[...] * hf
    oc_r[...] = q_r[...] * hc

  return pl.pallas_call(
      body, grid=(_GRID,),
      in_specs=[_rspec(True), _rspec(), _rspec(cols=1),
                _wspec(D, D), _wspec(1, D), _wspec(D, D), _wspec(1, D)],
      out_specs=[_rspec(), _rspec()],
      out_shape=[jax.ShapeDtypeStruct((NP, D), _f32),
                 jax.ShapeDtypeStruct((NP, D), _f32)])(
                     S0, g0, q, Wf, bf, Wc, bc)


def _tc_caus(S1f, S1c, g1f, g1c, q, Wf1, bf1, Wc1, bc1, Wsm, bsm):
  def body(sf_r, sc_r, gf_r, gc_r, q_r, wf_r, bf_r, wc_r, bc_r, wsm_r, bsm_r,
           xe_r, nc_r, p1_r, p2_r):
    q = q_r[...]
    xe_r[...] = jax.nn.relu(
        jnp.dot(q * (sf_r[0] + sf_r[1] + gf_r[...]), wf_r[...],
                preferred_element_type=_f32) + bf_r[...])
    h2c = jax.nn.relu(
        jnp.dot(q * (sc_r[0] + sc_r[1] + gc_r[...]), wc_r[...],
                preferred_element_type=_f32) + bc_r[...])
    sm = jnp.dot(h2c, wsm_r[...], preferred_element_type=_f32) + bsm_r[...]
    nc_r[...] = jax.nn.sigmoid(sm[:, 0:1])
    p1_r[...] = jnp.broadcast_to(sm[:, 1:2], (_RB, 16))
    p2_r[...] = jnp.broadcast_to(sm[:, 2:3], (_RB, 16))

  return pl.pallas_call(
      body, grid=(_GRID,),
      in_specs=[_rspec(True), _rspec(True), _rspec(), _rspec(),
                _rspec(cols=1),
                _wspec(D, D), _wspec(1, D), _wspec(D, D), _wspec(1, D),
                _wspec(D, 3), _wspec(1, 3)],
      out_specs=[_rspec(), _rspec(cols=1), _rspec(cols=16), _rspec(cols=16)],
      out_shape=[jax.ShapeDtypeStruct((NP, D), _f32),
                 jax.ShapeDtypeStruct((NP, 1), _f32),
                 jax.ShapeDtypeStruct((NP, 16), _f32),
                 jax.ShapeDtypeStruct((NP, 16), _f32)])(
                     S1f, S1c, g1f, g1c, q, Wf1, bf1, Wc1, bc1, Wsm, bsm)


def _tc_prep2(degb2, xe, nc):
  def body(d_r, xe_r, nc_r, qb_r, gb_r):
    qb = lax.rsqrt(d_r[0, :, 0:1] + d_r[1, :, 0:1] + 1.0)
    qb_r[...] = qb
    gb_r[...] = qb * (xe_r[...] * nc_r[...])

  return pl.pallas_call(
      body, grid=(_GRID,),
      in_specs=[_rspec(True, 16), _rspec(), _rspec(cols=1)],
      out_specs=[_rspec(cols=1), _rspec()],
      out_shape=[jax.ShapeDtypeStruct((NP, 1), _f32),
                 jax.ShapeDtypeStruct((NP, D), _f32)])(degb2, xe, nc)


def _tc_conv1(Sb, gb, qb, nc, W, b):
  def body(s_r, g_r, q_r, nc_r, w_r, b_r, o_r):
    h = jax.nn.relu(
        jnp.dot(q_r[...] * (s_r[0] + s_r[1] + g_r[...]), w_r[...],
                preferred_element_type=_f32) + b_r[...])
    o_r[...] = q_r[...] * (h * nc_r[...])

  return pl.pallas_call(
      body, grid=(_GRID,),
      in_specs=[_rspec(True), _rspec(), _rspec(cols=1), _rspec(cols=1),
                _wspec(D, D), _wspec(1, D)],
      out_specs=_rspec(),
      out_shape=jax.ShapeDtypeStruct((NP, D), _f32))(Sb, gb, qb, nc, W, b)


def _tc_conv_last(Sb, gb, qb, W, b):
  def body(s_r, g_r, q_r, w_r, b_r, o_r):
    o_r[...] = jax.nn.relu(
        jnp.dot(q_r[...] * (s_r[0] + s_r[1] + g_r[...]), w_r[...],
                preferred_element_type=_f32) + b_r[...])

  return pl.pallas_call(
      body, grid=(_GRID,),
      in_specs=[_rspec(True), _rspec(), _rspec(cols=1),
                _wspec(D, D), _wspec(1, D)],
      out_specs=_rspec(),
      out_shape=jax.ShapeDtypeStruct((NP, D), _f32))(Sb, gb, qb, W, b)


def _tc_pred(pooled, cnt2, W_pred, b_pred):
  def body(p_r, c_r, w_r, b_r, o_r):
    sums = p_r[0, :G, :] + p_r[1, :G, :]
    cnt = c_r[0, :G, 0:1] + c_r[1, :G, 0:1]
    hg = sums / jnp.maximum(cnt, 1.0)
    o_r[...] = jnp.dot(hg, w_r[...], preferred_element_type=_f32) + b_r[...]

  return pl.pallas_call(
      body,
      out_shape=jax.ShapeDtypeStruct((G, 10), _f32))(
          pooled, cnt2, W_pred, b_pred)


def kernel(x, edge_index, batch, W_f0, b_f0, W_f1, b_f1, W_c0, b_c0, W_c1,
           b_c1, W_node, b_node, W_edge, b_edge, W_b0, b_b0, W_b1, b_b1,
           W_pred, b_pred):
  src = edge_index[0].astype(_i32)
  dst = edge_index[1].astype(_i32)
  src2 = src.reshape(E // C, C)
  dst2 = dst.reshape(E // C, C)
  xp = jnp.pad(x, ((0, NP - N), (0, 0)))
  batchp = jnp.pad(batch.astype(_i32), (0, NP - N), constant_values=G)
  batch2 = batchp.reshape(NP // C, C)

  deg2, cnt2 = _sc_stats(dst2, batch2)
  q, g0 = _tc_prep(deg2, xp)
  S0 = _sc_spmm_half(g0, src2, dst2)
  g1f, g1c = _tc_conv2(S0, g0, q,
                       W_f0, b_f0.reshape(1, D), W_c0, b_c0.reshape(1, D))
  S1f = _sc_spmm_half(g1f, src2, dst2)
  S1c = _sc_spmm_half(g1c, src2, dst2)
  Wsm = jnp.concatenate([W_node, W_edge[:D], W_edge[D:]], axis=1)
  bsm = jnp.stack([b_node[0], jnp.zeros((), _f32), b_edge[0]]).reshape(1, 3)
  xe, nc, p1b, p2b = _tc_caus(S1f, S1c, g1f, g1c, q,
                              W_f1, b_f1.reshape(1, D), W_c1,
                              b_c1.reshape(1, D), Wsm, bsm)
  ec16, degb2 = _sc_edge(src2, dst2, p1b, p2b)
  qb, gb0 = _tc_prep2(degb2, xe, nc)
  Sb0 = _sc_spmm_half(gb0, src2, dst2, ec16)
  gb1 = _tc_conv1(Sb0, gb0, qb, nc, W_b0, b_b0.reshape(1, D))
  Sb1 = _sc_spmm_half(gb1, src2, dst2, ec16)
  h2 = _tc_conv_last(Sb1, gb1, qb, W_b1, b_b1.reshape(1, D))
  pooled = _sc_pool(h2, batchp)
  return _tc_pred(pooled, cnt2, W_pred, b_pred.reshape(1, 10))


# speedup vs baseline: 27.7256x; 1.0728x over previous
import functools

import jax
import jax.numpy as jnp
from jax import lax
from jax.experimental import pallas as pl
from jax.experimental.pallas import tpu as pltpu
from jax.experimental.pallas import tpu_sc as plsc

N = 10000
NP = 10240
E = 320000
D = 128
G = 64
GP = 128

NC = 2
NS = 16
NW = NC * NS
C = 40

E_SC = E // NC
E_TILE = E_SC // NS
NCH = E_TILE // C
E_TILE_F = E // NS
NCH_F = E_TILE_F // C
RTS = NP // NS
RTW = NP // NW

_MESH = plsc.VectorSubcoreMesh(
    core_axis_name="c", subcore_axis_name="s", num_cores=NC, num_subcores=NS)
_SC_PARAMS = pltpu.CompilerParams(use_tc_tiling_on_sc=False)

_f32 = jnp.float32
_i32 = jnp.int32


def _zero_rows(acc, zbuf, total):
  off = 0
  while off < total:
    n = min(C, total - off)
    pltpu.sync_copy(zbuf.at[pl.ds(0, n)], acc.at[pl.ds(off, n)])
    off += n


def _fill(ref, rows, cols, val, lead=None):
  v = jnp.full((16,), val, _f32)
  cg = cols // 16

  def body(i, carry):
    r = i // cg
    c = i % cg
    if lead is None:
      ref[r, pl.ds(c * 16, 16)] = v
    else:
      ref[lead, r, pl.ds(c * 16, 16)] = v
    return carry

  lax.fori_loop(0, rows * cg, body, 0)


def _sc_stats(dst, batch):
  @functools.partial(
      pl.kernel, mesh=_MESH, compiler_params=_SC_PARAMS,
      out_type=(jax.ShapeDtypeStruct((NC, NP, 16), _f32),
                jax.ShapeDtypeStruct((NC, GP, 16), _f32)),
      scratch_types=[
          pltpu.VMEM((NCH, C), _i32),
          pltpu.VMEM((RTW // C, C), _i32),
          pltpu.VMEM((C, 16), _f32),
      ] + [pltpu.SemaphoreType.DMA] * _NB + [
          pltpu.VMEM_SHARED((NP, 16), _f32),
          pltpu.VMEM_SHARED((GP, 16), _f32),
      ])
  def k(dst_h, batch_h, deg_o, cnt_o, didx_all, bidx_all, buf, *sems_acc):
    ssems = sems_acc[:_NB]
    accd, accc = sems_acc[_NB], sems_acc[_NB + 1]
    cid = lax.axis_index("c")
    sid = lax.axis_index("s")
    wid = cid * NS + sid
    _fill(buf, C, 16, 0.0)
    for j in range(RTS // C):
      pltpu.sync_copy(buf, accd.at[pl.ds(sid * RTS + j * C, C)])

    @pl.when(sid == 0)
    def _():
      _zero_rows(accc, buf, GP)

    cbase = (cid * E_SC + sid * E_TILE) // C
    pltpu.sync_copy(dst_h.at[pl.ds(cbase, NCH)], didx_all)
    pltpu.sync_copy(batch_h.at[pl.ds((wid * RTW) // C, RTW // C)], bidx_all)
    _fill(buf, C, 16, 1.0)
    plsc.subcore_barrier()

    def step(p, carry):
      for b in range(_NB):
        j = p * _NB + b

        @pl.when(j < NCH)
        def _(j=j, b=b):
          @pl.when(j >= _NB)
          def _():
            pltpu.make_async_copy(
                buf, accd.at[didx_all.at[j - _NB]], ssems[b]).wait()
          pltpu.async_copy(buf, accd.at[didx_all.at[j]], ssems[b], add=True)

      return carry

    lax.fori_loop(0, (NCH + _NB - 1) // _NB, step, 0)
    for kk in range(_NB):
      c = NCH - _NB + kk
      pltpu.make_async_copy(
          buf, accd.at[didx_all.at[c]], ssems[c % _NB]).wait()

    for j in range(RTW // C):
      pltpu.sync_copy(buf, accc.at[bidx_all.at[j]], add=True)

    plsc.subcore_barrier()
    pltpu.sync_copy(accd.at[pl.ds(sid * RTS, RTS)],
                    deg_o.at[cid, pl.ds(sid * RTS, RTS)])

    @pl.when(sid == 0)
    def _():
      pltpu.sync_copy(accc, cnt_o.at[cid])

  return k(dst, batch)


_NB = 5
_LA = 4


def _pipe(g_h, sidx_all, didx_all, rows, gsems, ssems, acc, nch,
          scale_fn=None, pre_fn=None):
  for b in range(_LA):
    pltpu.async_copy(g_h.at[sidx_all.at[b]], rows.at[b], gsems[b])
    if pre_fn is not None:
      pre_fn(b, b)

  def step(p, carry):
    for b in range(_NB):
      j = p * _NB + b

      @pl.when(j < nch)
      def _(j=j, b=b):
        pltpu.make_async_copy(
            g_h.at[sidx_all.at[j]], rows.at[b], gsems[b]).wait()
        if scale_fn is not None:
          scale_fn(j, b)
        pltpu.async_copy(rows.at[b], acc.at[didx_all.at[j]], ssems[b],
                         add=True)

      jn = j + _LA
      bn = (b + _LA) % _NB

      @pl.when(jnp.logical_and(jn < nch, jn >= _NB))
      def _(jn=jn, bn=bn):
        pltpu.make_async_copy(
            rows.at[bn], acc.at[didx_all.at[jn - _NB]], ssems[bn]).wait()
        pltpu.async_copy(g_h.at[sidx_all.at[jn]], rows.at[bn], gsems[bn])
        if pre_fn is not None:
          pre_fn(jn, bn)

      @pl.when(jnp.logical_and(jn < nch, jn < _NB))
      def _(jn=jn, bn=bn):
        pltpu.async_copy(g_h.at[sidx_all.at[jn]], rows.at[bn], gsems[bn])
        if pre_fn is not None:
          pre_fn(jn, bn)

    return carry

  lax.fori_loop(0, (nch + _NB - 1) // _NB, step, 0)
  for kk in range(_NB):
    c = nch - _NB + kk
    pltpu.make_async_copy(
        rows.at[c % _NB], acc.at[didx_all.at[c]], ssems[c % _NB]).wait()


def _spmm_scratch(nch):
  return [
      pltpu.VMEM((nch, C), _i32),
      pltpu.VMEM((nch, C), _i32),
      pltpu.VMEM((_NB, C, D), _f32),
  ] + [pltpu.SemaphoreType.DMA] * (2 * _NB) + [
      pltpu.VMEM_SHARED((NP, D), _f32),
  ]


def _sc_spmm_half(g, src2, dst2, ec16=None):
  weighted = ec16 is not None
  scratch = _spmm_scratch(NCH) + (
      [pltpu.VMEM((_NB, C, 16), _f32)] +
      [pltpu.SemaphoreType.DMA] * _NB if weighted else [])

  @functools.partial(
      pl.kernel, mesh=_MESH, compiler_params=_SC_PARAMS,
      out_type=jax.ShapeDtypeStruct((NC, NP, D), _f32),
      scratch_types=scratch)
  def k(*args):
    if weighted:
      (g_h, src_h, dst_h, ec_h, out_h, sidx_all, didx_all, rows,
       *rest) = args
      ecv = rest[2 * _NB + 1]
      esems = rest[2 * _NB + 2:]
    else:
      (g_h, src_h, dst_h, out_h, sidx_all, didx_all, rows, *rest) = args
    gsems, ssems = rest[:_NB], rest[_NB:2 * _NB]
    acc = rest[2 * _NB]
    cid = lax.axis_index("c")
    sid = lax.axis_index("s")
    _fill(rows, C, D, 0.0, lead=0)
    for j in range(RTS // C):
      pltpu.sync_copy(rows.at[0], acc.at[pl.ds(sid * RTS + j * C, C)])
    cbase = (cid * E_SC + sid * E_TILE) // C
    pltpu.sync_copy(src_h.at[pl.ds(cbase, NCH)], sidx_all)
    pltpu.sync_copy(dst_h.at[pl.ds(cbase, NCH)], didx_all)
    plsc.subcore_barrier()

    if weighted:
      ebase = cid * E_SC + sid * E_TILE

      def pre(jn, bn):
        pltpu.async_copy(ec_h.at[pl.ds(ebase + jn * C, C)], ecv.at[bn],
                         esems[bn])

      def scale(j, b):
        pltpu.make_async_copy(
            ec_h.at[pl.ds(0, C)], ecv.at[b], esems[b]).wait()

        def srow(r8, c2):
          for u in range(8):
            r = r8 * 8 + u
            ev = ecv[b, r, :]
            for q in range(D // 16):
              rows[b, r, pl.ds(q * 16, 16)] = (
                  rows[b, r, pl.ds(q * 16, 16)] * ev)
          return c2

        lax.fori_loop(0, C // 8, srow, 0)
    else:
      scale = None
      pre = None

    _pipe(g_h, sidx_all, didx_all, rows, gsems, ssems, acc, NCH, scale, pre)
    plsc.subcore_barrier()
    for j in range(RTS // C):
      r0 = sid * RTS + j * C
      pltpu.sync_copy(acc.at[pl.ds(r0, C)], out_h.at[cid, pl.ds(r0, C)])

  if weighted:
    return k(g, src2, dst2, ec16)
  return k(g, src2, dst2)


def _sc_edge(src2, dst2, p1b, p2b):
  @functools.partial(
      pl.kernel, mesh=_MESH, compiler_params=_SC_PARAMS,
      out_type=(jax.ShapeDtypeStruct((E, 16), _f32),
                jax.ShapeDtypeStruct((NC, NP, 16), _f32)),
      scratch_types=[
          pltpu.VMEM((NCH, C), _i32),
          pltpu.VMEM((NCH, C), _i32),
          pltpu.VMEM((_NB, C, 16), _f32),
          pltpu.VMEM((_NB, C, 16), _f32),
          pltpu.VMEM((_NB, C, 16), _f32),
          pltpu.VMEM((C, 16), _f32),
      ] + [pltpu.SemaphoreType.DMA] * (3 * _NB) + [
          pltpu.VMEM_SHARED((NP, 16), _f32),
      ])
  def k(src_h, dst_h, p1b_h, p2b_h, ec_o, degb_o,
        sidx_all, didx_all, ra, rb, ecv, zbuf, *sems_acc):
    gsems = sems_acc[:_NB]
    ssems = sems_acc[_NB:2 * _NB]
    osems = sems_acc[2 * _NB:3 * _NB]
    acc = sems_acc[3 * _NB]
    cid = lax.axis_index("c")
    sid = lax.axis_index("s")
    _fill(zbuf, C, 16, 0.0)
    for j in range(RTS // C):
      pltpu.sync_copy(zbuf, acc.at[pl.ds(sid * RTS + j * C, C)])
    cbase = (cid * E_SC + sid * E_TILE) // C
    pltpu.sync_copy(src_h.at[pl.ds(cbase, NCH)], sidx_all)
    pltpu.sync_copy(dst_h.at[pl.ds(cbase, NCH)], didx_all)
    plsc.subcore_barrier()

    ebase = cid * E_SC + sid * E_TILE

    def start_gather(j, b):
      pltpu.async_copy(p1b_h.at[sidx_all.at[j]], ra.at[b], gsems[b])
      pltpu.async_copy(p2b_h.at[didx_all.at[j]], rb.at[b], gsems[b])

    for b in range(_LA):
      start_gather(b, b)

    def step(p, carry):
      for b in range(_NB):
        j = p * _NB + b

        @pl.when(j < NCH)
        def _(j=j, b=b):
          pltpu.make_async_copy(
              p1b_h.at[sidx_all.at[j]], ra.at[b], gsems[b]).wait()
          pltpu.make_async_copy(
              p2b_h.at[didx_all.at[j]], rb.at[b], gsems[b]).wait()

          @pl.when(j >= _NB)
          def _():
            pltpu.make_async_copy(
                ecv.at[b], acc.at[didx_all.at[j - _NB]], ssems[b]).wait()
            pltpu.make_async_copy(
                ecv.at[b], ec_o.at[pl.ds(0, C)], osems[b]).wait()

          def erow(r8, c2):
            for u in range(8):
              r = r8 * 8 + u
              z = ra[b, r, :] + rb[b, r, :]
              ecv[b, r, :] = 1.0 / (1.0 + jnp.exp(-z))
            return c2

          lax.fori_loop(0, C // 8, erow, 0)
          pltpu.async_copy(ecv.at[b], acc.at[didx_all.at[j]], ssems[b],
                           add=True)
          pltpu.async_copy(ecv.at[b], ec_o.at[pl.ds(ebase + j * C, C)],
                           osems[b])

        jn = j + _LA
        bn = (b + _LA) % _NB

        @pl.when(jn < NCH)
        def _(jn=jn, bn=bn):
          start_gather(jn, bn)

      return carry

    lax.fori_loop(0, (NCH + _NB - 1) // _NB, step, 0)
    for kk in range(_NB):
      c = NCH - _NB + kk
      pltpu.make_async_copy(
          ecv.at[c % _NB], acc.at[didx_all.at[c]], ssems[c % _NB]).wait()
      pltpu.make_async_copy(
          ecv.at[c % _NB], ec_o.at[pl.ds(0, C)], osems[c % _NB]).wait()
    plsc.subcore_barrier()
    pltpu.sync_copy(acc.at[pl.ds(sid * RTS, RTS)],
                    degb_o.at[cid, pl.ds(sid * RTS, RTS)])

  return k(src2, dst2, p1b, p2b)


def _sc_pool(h, batch):
  @functools.partial(
      pl.kernel, mesh=_MESH, compiler_params=_SC_PARAMS,
      out_type=jax.ShapeDtypeStruct((NC, GP, D), _f32),
      scratch_types=[
          pltpu.VMEM((C,), _i32),
          pltpu.VMEM((C, D), _f32),
          pltpu.VMEM((C, D), _f32),
          pltpu.VMEM_SHARED((GP, D), _f32),
      ])
  def k(h_h, batch_h, out_h, bidx, rows, zbuf, acc):
    cid = lax.axis_index("c")
    sid = lax.axis_index("s")
    wid = cid * NS + sid

    @pl.when(sid == 0)
    def _():
      _fill(zbuf, C, D, 0.0)
      _zero_rows(acc, zbuf, GP)

    plsc.subcore_barrier()
    nbase = wid * RTW

    def step(j, carry):
      off = nbase + j * C
      pltpu.sync_copy(batch_h.at[pl.ds(off, C)], bidx)
      pltpu.sync_copy(h_h.at[pl.ds(off, C)], rows)
      pltpu.sync_copy(rows, acc.at[bidx], add=True)
      return carry

    lax.fori_loop(0, RTW // C, step, 0)
    plsc.subcore_barrier()

    @pl.when(sid == 0)
    def _():
      pltpu.sync_copy(acc, out_h.at[cid])

  return k(h, batch)


_RB = 512
_GRID = NP // _RB


def _rspec(shape3=False, cols=D):
  if shape3:
    return pl.BlockSpec((NC, _RB, cols), lambda j: (0, j, 0))
  return pl.BlockSpec((_RB, cols), lambda j: (j, 0))


def _wspec(r, c):
  return pl.BlockSpec((r, c), lambda j: (0, 0))


def _tc_prep(deg2, x):
  def body(d_r, x_r, q_r, g_r):
    deg = d_r[0, :, 0:1] + d_r[1, :, 0:1] + 1.0
    q = lax.rsqrt(deg)
    q_r[...] = q
    g_r[...] = q * x_r[...]

  return pl.pallas_call(
      body, grid=(_GRID,),
      in_specs=[_rspec(True, 16), _rspec()],
      out_specs=[_rspec(cols=1), _rspec()],
      out_shape=[jax.ShapeDtypeStruct((NP, 1), _f32),
                 jax.ShapeDtypeStruct((NP, D), _f32)])(deg2, x)


def _tc_conv2(S0, g0, q, Wf, bf, Wc, bc):
  def body(s_r, g_r, q_r, wf_r, bf_r, wc_r, bc_r, of_r, oc_r):
    t = q_r[...] * (s_r[0] + s_r[1] + g_r[...])
    hf = jax.nn.relu(jnp.dot(t, wf_r[...], preferred_element_type=_f32)
                     + bf_r[...])
    hc = jax.nn.relu(jnp.dot(t, wc_r[...], preferred_element_type=_f32)
                     + bc_r[...])
    of_r[...] = q_r[...] * hf
    oc_r[...] = q_r[...] * hc

  return pl.pallas_call(
      body, grid=(_GRID,),
      in_specs=[_rspec(True), _rspec(), _rspec(cols=1),
                _wspec(D, D), _wspec(1, D), _wspec(D, D), _wspec(1, D)],
      out_specs=[_rspec(), _rspec()],
      out_shape=[jax.ShapeDtypeStruct((NP, D), _f32),
                 jax.ShapeDtypeStruct((NP, D), _f32)])(
                     S0, g0, q, Wf, bf, Wc, bc)


def _tc_caus(S1f, S1c, g1f, g1c, q, Wf1, bf1, Wc1, bc1, Wsm, bsm):
  def body(sf_r, sc_r, gf_r, gc_r, q_r, wf_r, bf_r, wc_r, bc_r, wsm_r, bsm_r,
           xe_r, nc_r, p1_r, p2_r):
    q = q_r[...]
    xe_r[...] = jax.nn.relu(
        jnp.dot(q * (sf_r[0] + sf_r[1] + gf_r[...]), wf_r[...],
                preferred_element_type=_f32) + bf_r[...])
    h2c = jax.nn.relu(
        jnp.dot(q * (sc_r[0] + sc_r[1] + gc_r[...]), wc_r[...],
                preferred_element_type=_f32) + bc_r[...])
    sm = jnp.dot(h2c, wsm_r[...], preferred_element_type=_f32) + bsm_r[...]
    nc_r[...] = jax.nn.sigmoid(sm[:, 0:1])
    p1_r[...] = jnp.broadcast_to(sm[:, 1:2], (_RB, 16))
    p2_r[...] = jnp.broadcast_to(sm[:, 2:3], (_RB, 16))

  return pl.pallas_call(
      body, grid=(_GRID,),
      in_specs=[_rspec(True), _rspec(True), _rspec(), _rspec(),
                _rspec(cols=1),
                _wspec(D, D), _wspec(1, D), _wspec(D, D), _wspec(1, D),
                _wspec(D, 3), _wspec(1, 3)],
      out_specs=[_rspec(), _rspec(cols=1), _rspec(cols=16), _rspec(cols=16)],
      out_shape=[jax.ShapeDtypeStruct((NP, D), _f32),
                 jax.ShapeDtypeStruct((NP, 1), _f32),
                 jax.ShapeDtypeStruct((NP, 16), _f32),
                 jax.ShapeDtypeStruct((NP, 16), _f32)])(
                     S1f, S1c, g1f, g1c, q, Wf1, bf1, Wc1, bc1, Wsm, bsm)


def _tc_prep2(degb2, xe, nc):
  def body(d_r, xe_r, nc_r, qb_r, gb_r):
    qb = lax.rsqrt(d_r[0, :, 0:1] + d_r[1, :, 0:1] + 1.0)
    qb_r[...] = qb
    gb_r[...] = qb * (xe_r[...] * nc_r[...])

  return pl.pallas_call(
      body, grid=(_GRID,),
      in_specs=[_rspec(True, 16), _rspec(), _rspec(cols=1)],
      out_specs=[_rspec(cols=1), _rspec()],
      out_shape=[jax.ShapeDtypeStruct((NP, 1), _f32),
                 jax.ShapeDtypeStruct((NP, D), _f32)])(degb2, xe, nc)


def _tc_conv1(Sb, gb, qb, nc, W, b):
  def body(s_r, g_r, q_r, nc_r, w_r, b_r, o_r):
    h = jax.nn.relu(
        jnp.dot(q_r[...] * (s_r[0] + s_r[1] + g_r[...]), w_r[...],
                preferred_element_type=_f32) + b_r[...])
    o_r[...] = q_r[...] * (h * nc_r[...])

  return pl.pallas_call(
      body, grid=(_GRID,),
      in_specs=[_rspec(True), _rspec(), _rspec(cols=1), _rspec(cols=1),
                _wspec(D, D), _wspec(1, D)],
      out_specs=_rspec(),
      out_shape=jax.ShapeDtypeStruct((NP, D), _f32))(Sb, gb, qb, nc, W, b)


def _tc_conv_last(Sb, gb, qb, W, b):
  def body(s_r, g_r, q_r, w_r, b_r, o_r):
    o_r[...] = jax.nn.relu(
        jnp.dot(q_r[...] * (s_r[0] + s_r[1] + g_r[...]), w_r[...],
                preferred_element_type=_f32) + b_r[...])

  return pl.pallas_call(
      body, grid=(_GRID,),
      in_specs=[_rspec(True), _rspec(), _rspec(cols=1),
                _wspec(D, D), _wspec(1, D)],
      out_specs=_rspec(),
      out_shape=jax.ShapeDtypeStruct((NP, D), _f32))(Sb, gb, qb, W, b)


def _tc_pred(pooled, cnt2, W_pred, b_pred):
  def body(p_r, c_r, w_r, b_r, o_r):
    sums = p_r[0, :G, :] + p_r[1, :G, :]
    cnt = c_r[0, :G, 0:1] + c_r[1, :G, 0:1]
    hg = sums / jnp.maximum(cnt, 1.0)
    o_r[...] = jnp.dot(hg, w_r[...], preferred_element_type=_f32) + b_r[...]

  return pl.pallas_call(
      body,
      out_shape=jax.ShapeDtypeStruct((G, 10), _f32))(
          pooled, cnt2, W_pred, b_pred)


def kernel(x, edge_index, batch, W_f0, b_f0, W_f1, b_f1, W_c0, b_c0, W_c1,
           b_c1, W_node, b_node, W_edge, b_edge, W_b0, b_b0, W_b1, b_b1,
           W_pred, b_pred):
  src = edge_index[0].astype(_i32)
  dst = edge_index[1].astype(_i32)
  src2 = src.reshape(E // C, C)
  dst2 = dst.reshape(E // C, C)
  xp = jnp.pad(x, ((0, NP - N), (0, 0)))
  batchp = jnp.pad(batch.astype(_i32), (0, NP - N), constant_values=G)
  batch2 = batchp.reshape(NP // C, C)

  deg2, cnt2 = _sc_stats(dst2, batch2)
  q, g0 = _tc_prep(deg2, xp)
  S0 = _sc_spmm_half(g0, src2, dst2)
  g1f, g1c = _tc_conv2(S0, g0, q,
                       W_f0, b_f0.reshape(1, D), W_c0, b_c0.reshape(1, D))
  S1f = _sc_spmm_half(g1f, src2, dst2)
  S1c = _sc_spmm_half(g1c, src2, dst2)
  Wsm = jnp.concatenate([W_node, W_edge[:D], W_edge[D:]], axis=1)
  bsm = jnp.stack([b_node[0], jnp.zeros((), _f32), b_edge[0]]).reshape(1, 3)
  xe, nc, p1b, p2b = _tc_caus(S1f, S1c, g1f, g1c, q,
                              W_f1, b_f1.reshape(1, D), W_c1,
                              b_c1.reshape(1, D), Wsm, bsm)
  ec16, degb2 = _sc_edge(src2, dst2, p1b, p2b)
  qb, gb0 = _tc_prep2(degb2, xe, nc)
  Sb0 = _sc_spmm_half(gb0, src2, dst2, ec16)
  gb1 = _tc_conv1(Sb0, gb0, qb, nc, W_b0, b_b0.reshape(1, D))
  Sb1 = _sc_spmm_half(gb1, src2, dst2, ec16)
  h2 = _tc_conv_last(Sb1, gb1, qb, W_b1, b_b1.reshape(1, D))
  pooled = _sc_pool(h2, batchp)
  return _tc_pred(pooled, cnt2, W_pred, b_pred.reshape(1, 10))
